# baseline (speedup 1.0000x reference)
import functools
import numpy as np
import jax
import jax.numpy as jnp
from jax import lax
from jax.experimental import pallas as pl
from jax.experimental.pallas import tpu as pltpu
from jax.experimental.pallas import tpu_sc as plsc

N0 = 20000
W = 16
NA = 1280
NT = W * NA
NTAB = 24576
NCH = NA // 16
NEG_INF = np.float32(-np.inf)
IMAX = np.int32(2147483647)
IMIN = np.int32(-2147483648)


def _iota():
    return lax.iota(jnp.int32, 16)


def _sumi(vec):
    return jnp.sum(vec.astype(jnp.float32)).astype(jnp.int32)


def _ext(vec, idx):
    return _sumi(jnp.where(_iota() == idx, vec, 0))


def _ln(x):
    u = lax.bitcast_convert_type(x, jnp.int32)
    e = ((u >> 23) & 0xFF) - 127
    m = lax.bitcast_convert_type((u & 0x007FFFFF) | 0x3F800000, jnp.float32)
    big = m > jnp.float32(1.4142135)
    m = jnp.where(big, m * jnp.float32(0.5), m)
    e = (e + jnp.where(big, 1, 0)).astype(jnp.float32)
    z = (m - 1.0) / (m + 1.0)
    z2 = z * z
    p = z * (2.0 + z2 * (2.0 / 3.0 + z2 * (0.4 + z2 * (2.0 / 7.0 + z2 * (2.0 / 9.0)))))
    return e * jnp.float32(0.6931471805599453) + p


def _orderable(k):
    u = lax.bitcast_convert_type(k, jnp.int32)
    return u ^ ((u >> 31) & 0x7FFFFFFF)


def _body(a_hbm, gtc_hbm, wh_hbm, gpos_hbm, gneg_hbm, logt_hbm,
          off_hbm, lab_hbm,
          av, maskv, lrankv, pamaxv, pargv, iouv, labelv, keyv, selv,
          gtcv, whv, gmaxv, gtmaxlocv, cntv, histv, gslv, logtv, offv,
          histall, gmaxall, cntall,
          sh_gtmax, sh_cnt, sh_hist, sh_kept):
    wid = lax.axis_index("s")
    base = wid * NA
    iot = _iota()
    onesi = jnp.ones((16,), jnp.int32)
    zerosi = jnp.zeros((16,), jnp.int32)

    for f in range(4):
        pltpu.sync_copy(a_hbm.at[pl.ds(f * NT + base, NA)], av.at[pl.ds(f * NA, NA)])
    pltpu.sync_copy(gtc_hbm, gtcv)
    pltpu.sync_copy(wh_hbm, whv)
    wv = whv[pl.ds(0, 16)]
    hv = whv[pl.ds(16, 16)]

    def p1_chunk(c, car):
        cmask, rbase = car
        ax1 = av[pl.ds(0 * NA + c * 16, 16)]
        ay1 = av[pl.ds(1 * NA + c * 16, 16)]
        ax2 = av[pl.ds(2 * NA + c * 16, 16)]
        ay2 = av[pl.ds(3 * NA + c * 16, 16)]
        mx = (ax1 >= 0.0) & (ay1 >= 0.0) & (ax2 <= wv) & (ay2 <= hv)
        mi = jnp.where(mx, 1, 0)
        maskv[pl.ds(c * 16, 16)] = mi
        mf = jnp.where(mx, 1.0, 0.0)
        cs = (plsc.cumsum(mf) - mf).astype(jnp.int32)
        lrankv[pl.ds(c * 16, 16)] = rbase + cs
        area_a = (ax2 - ax1) * (ay2 - ay1)

        def p1_gt(j, icar):
            pamax, parg = icar
            jf = jnp.full((16,), j, jnp.int32)
            gx1 = plsc.load_gather(gtcv, [zerosi, jf])
            gy1 = plsc.load_gather(gtcv, [onesi, jf])
            gx2 = plsc.load_gather(gtcv, [onesi + onesi, jf])
            gy2 = plsc.load_gather(gtcv, [onesi + onesi + onesi, jf])
            ix1 = jnp.maximum(ax1, gx1)
            iy1 = jnp.maximum(ay1, gy1)
            ix2 = jnp.minimum(ax2, gx2)
            iy2 = jnp.minimum(ay2, gy2)
            iw = jnp.maximum(ix2 - ix1, 0.0)
            ih = jnp.maximum(iy2 - iy1, 0.0)
            inter = iw * ih
            area_b = (gx2 - gx1) * (gy2 - gy1)
            iou = jnp.where(mx, inter / (area_a + area_b - inter), NEG_INF)
            iouv[pl.ds(j * NA + c * 16, 16)] = iou
            better = iou > pamax
            pamax = jnp.where(better, iou, pamax)
            parg = jnp.where(better, jf, parg)
            return pamax, parg

        pamax0 = jnp.full((16,), NEG_INF, jnp.float32)
        pamax, parg = lax.fori_loop(0, 32, p1_gt, (pamax0, zerosi))
        pamaxv[pl.ds(c * 16, 16)] = pamax
        pargv[pl.ds(c * 16, 16)] = parg
        nm = _sumi(mi)
        return cmask + nm, rbase + nm

    ninf16 = jnp.full((16,), NEG_INF, jnp.float32)
    cmask, _ = lax.fori_loop(
        0, NCH, p1_chunk, (jnp.int32(0), jnp.int32(0)))

    def rowmax_j(j, car):
        glo, ghi = car

        def rm_c(c, acc):
            return jnp.maximum(acc, iouv[pl.ds(j * NA + c * 16, 16)])

        s = jnp.max(lax.fori_loop(0, NCH, rm_c, ninf16))
        sb = jnp.full((16,), s, jnp.float32)
        glo = jnp.where(iot == j, jnp.maximum(glo, sb), glo)
        ghi = jnp.where(iot == (j - 16), jnp.maximum(ghi, sb), ghi)
        return glo, ghi

    glo, ghi = lax.fori_loop(0, 32, rowmax_j, (ninf16, ninf16))
    gtmaxlocv[pl.ds(0, 16)] = glo
    gtmaxlocv[pl.ds(16, 16)] = ghi
    pltpu.sync_copy(gtmaxlocv, sh_gtmax.at[wid])
    plsc.subcore_barrier()

    pltpu.sync_copy(sh_gtmax, gmaxall)

    def gmax_w(w, car):
        glo, ghi = car
        return (jnp.maximum(glo, gmaxall[w, pl.ds(0, 16)]),
                jnp.maximum(ghi, gmaxall[w, pl.ds(16, 16)]))

    glo, ghi = lax.fori_loop(0, W, gmax_w, (ninf16, ninf16))
    gmaxv[pl.ds(0, 16)] = glo
    gmaxv[pl.ds(16, 16)] = ghi

    def p2_chunk(c, car):
        cpos, cneg, spos, sneg = car
        pam = pamaxv[pl.ds(c * 16, 16)]
        mi = maskv[pl.ds(c * 16, 16)]
        mx = mi != 0
        lr = lrankv[pl.ds(c * 16, 16)]

        def p2_gt(j, acc):
            gmb = plsc.load_gather(gmaxv, [jnp.full((16,), j, jnp.int32)])
            eq = (iouv[pl.ds(j * NA + c * 16, 16)] == gmb) & mx
            return acc | jnp.where(eq, 1, 0)

        isgt = lax.fori_loop(0, 32, p2_gt, zerosi) != 0
        one = isgt | (mx & (pam >= jnp.float32(0.7)))
        neg = mx & (pam <= jnp.float32(0.3)) & jnp.logical_not(one)
        lab = jnp.where(one, 1, jnp.where(neg, 0, -1))
        labelv[pl.ds(c * 16, 16)] = lab

        ax1 = av[pl.ds(0 * NA + c * 16, 16)]
        ay1 = av[pl.ds(1 * NA + c * 16, 16)]
        ax2 = av[pl.ds(2 * NA + c * 16, 16)]
        ay2 = av[pl.ds(3 * NA + c * 16, 16)]
        parg = pargv[pl.ds(c * 16, 16)]
        gx1 = plsc.load_gather(gtcv, [zerosi, parg])
        gy1 = plsc.load_gather(gtcv, [onesi, parg])
        gx2 = plsc.load_gather(gtcv, [onesi + onesi, parg])
        gy2 = plsc.load_gather(gtcv, [onesi + onesi + onesi, parg])
        aw = ax2 - ax1
        ah = ay2 - ay1
        acx = ax1 + 0.5 * aw
        acy = ay1 + 0.5 * ah
        gw = gx2 - gx1
        gh = gy2 - gy1
        gcx = gx1 + 0.5 * gw
        gcy = gy1 + 0.5 * gh
        zf = jnp.zeros((16,), jnp.float32)
        offv[pl.ds(0 * NA + c * 16, 16)] = jnp.where(mx, (gcx - acx) / aw, zf)
        offv[pl.ds(1 * NA + c * 16, 16)] = jnp.where(mx, (gcy - acy) / ah, zf)
        offv[pl.ds(2 * NA + c * 16, 16)] = jnp.where(mx, _ln(gw / aw), zf)
        offv[pl.ds(3 * NA + c * 16, 16)] = jnp.where(mx, _ln(gh / ah), zf)

        onei = jnp.where(one, 1, 0)
        negi = jnp.where(neg, 1, 0)
        return (cpos + _sumi(onei), cneg + _sumi(negi),
                spos + _sumi(jnp.where(one, lr, 0)),
                sneg + _sumi(jnp.where(neg, lr, 0)))

    cpos, cneg, spos, sneg = lax.fori_loop(
        0, NCH, p2_chunk, (jnp.int32(0),) * 4)
    for f in range(4):
        pltpu.sync_copy(offv.at[pl.ds(f * NA, NA)], off_hbm.at[pl.ds(f * NT + base, NA)])
    cv = jnp.where(iot == 0, cmask, 0)
    cv = jnp.where(iot == 1, cpos, cv)
    cv = jnp.where(iot == 2, cneg, cv)
    cv = jnp.where(iot == 3, spos, cv)
    cv = jnp.where(iot == 4, sneg, cv)
    cntv[pl.ds(0, 16)] = cv
    pltpu.sync_copy(cntv, sh_cnt.at[wid])
    plsc.subcore_barrier()

    pltpu.sync_copy(sh_cnt, cntall)

    def red_w(w, car):
        rb_all, rb_my, npos, posb, nneg, negb, rsp, rsn = car
        row = cntall[w, pl.ds(0, 16)]
        cm = _ext(row, 0)
        cp = _ext(row, 1)
        cn = _ext(row, 2)
        sp = _ext(row, 3)
        sn = _ext(row, 4)
        before = jnp.where(w < wid, 1, 0)
        return (rb_all + cm, rb_my + before * cm, npos + cp,
                posb + before * cp, nneg + cn, negb + before * cn,
                rsp + cp * rb_all + sp, rsn + cn * rb_all + sn)

    (_, rankbase, npos, posbase, nneg, negbase, rsumpos, rsumneg) = lax.fori_loop(
        0, W, red_w, (jnp.int32(0),) * 8)

    rb8 = pl.multiple_of((rankbase >> 3) << 3, 8)
    rrem = rankbase - rb8
    pltpu.sync_copy(logt_hbm.at[pl.ds(rb8, NA + 8)], logtv)

    def run_select(target, K, g_hbm, selbase, ntgt, rsum):
        cond = (ntgt > K) & (rsum > 0)
        condi = jnp.where(cond, 1, 0)
        sb8 = pl.multiple_of((selbase >> 3) << 3, 8)
        srem = selbase - sb8
        pltpu.sync_copy(g_hbm.at[pl.ds(sb8, NA + 8)], gslv)

        def key_chunk(c, jb):
            lab = labelv[pl.ds(c * 16, 16)]
            sel = lab == target
            si = jnp.where(sel, 1, 0)
            sf = jnp.where(sel, 1.0, 0.0)
            jloc = jb + (plsc.cumsum(sf) - sf).astype(jnp.int32)
            gval = plsc.load_gather(gslv, [jloc + srem])
            lg = plsc.load_gather(logtv, [lrankv[pl.ds(c * 16, 16)] + rrem])
            k = gval + lg
            v = _orderable(k)
            act = sel & (condi != 0)
            keyv[pl.ds(c * 16, 16)] = jnp.where(act, v, IMAX)
            selv[pl.ds(c * 16, 16)] = jnp.where(act, 1, 0)
            return jb + _sumi(si)

        lax.fori_loop(0, NCH, key_chunk, jnp.int32(0))

        def radix_round(r, pc):
            P, C = pc
            shift = 24 - 8 * r

            def zero_h(h, _):
                histv[pl.ds(h * 16, 16)] = zerosi
                return 0

            lax.fori_loop(0, 16, zero_h, 0)

            def hist_chunk(c, _):
                v = keyv[pl.ds(c * 16, 16)]
                wbits = v ^ IMIN
                dig = lax.shift_right_logical(wbits, shift) & 255
                hb = lax.shift_right_logical(
                    lax.shift_right_logical(wbits, shift + 7), 1)
                act = (selv[pl.ds(c * 16, 16)] != 0) & (hb == P)
                plsc.addupdate_scatter(histv, [dig], onesi, mask=act)
                return 0

            lax.fori_loop(0, NCH, hist_chunk, 0)
            pltpu.sync_copy(histv, sh_hist.at[wid])
            plsc.subcore_barrier()

            pltpu.sync_copy(sh_hist, histall)

            def merge_w(w, _):
                def merge_h(h, _):
                    cur = jnp.where(w == 0, zerosi, histv[pl.ds(h * 16, 16)])
                    histv[pl.ds(h * 16, 16)] = cur + histall[w, pl.ds(h * 16, 16)]
                    return 0

                return lax.fori_loop(0, 16, merge_h, 0)

            lax.fori_loop(0, W, merge_w, 0)

            K1f = (K - C).astype(jnp.float32)

            def find_h(h, car):
                cum, found, bdig, cexcl = car
                accf = histv[pl.ds(h * 16, 16)].astype(jnp.float32)
                csf = plsc.cumsum(accf)
                cand = (cum.astype(jnp.float32) + csf) >= K1f
                idx = jnp.min(jnp.where(cand, iot.astype(jnp.float32), 16.0)).astype(jnp.int32)
                hit = (found == 0) & (idx < 16)
                csi = jnp.sum(jnp.where(iot == idx, csf - accf, 0.0)).astype(jnp.int32)
                bdig = jnp.where(hit, h * 16 + idx, bdig)
                cexcl = jnp.where(hit, cum + csi, cexcl)
                found = jnp.where(hit, 1, found)
                cum = cum + jnp.sum(jnp.where(iot == 15, csf, 0.0)).astype(jnp.int32)
                return cum, found, bdig, cexcl

            _, _, bdig, cexcl = lax.fori_loop(
                0, 16, find_h, (jnp.int32(0),) * 4)
            plsc.subcore_barrier()
            return (P << 8) | bdig, C + cexcl

        P, C = lax.fori_loop(0, 4, radix_round, (jnp.int32(0), jnp.int32(0)))

        Tv = P ^ IMIN

        def apply_chunk(c, kept):
            key = keyv[pl.ds(c * 16, 16)]
            sel = selv[pl.ds(c * 16, 16)] != 0
            rm = sel & (key > Tv)
            lab = labelv[pl.ds(c * 16, 16)]
            labelv[pl.ds(c * 16, 16)] = jnp.where(rm, -1, lab)
            return kept + _sumi(jnp.where(sel & (key <= Tv), 1, 0))

        kept = lax.fori_loop(0, NCH, apply_chunk, jnp.int32(0))
        return kept, cond

    kept_loc, condp = run_select(jnp.int32(1), jnp.int32(128), gpos_hbm,
                                 posbase, npos, rsumpos)
    cntv[pl.ds(0, 16)] = jnp.where(iot == 0, kept_loc, 0)
    pltpu.sync_copy(cntv, sh_kept.at[wid])
    plsc.subcore_barrier()
    pltpu.sync_copy(sh_kept, cntall)

    def kept_w(w, acc):
        return acc + _ext(cntall[w, pl.ds(0, 16)], 0)

    kept_tot = lax.fori_loop(0, W, kept_w, jnp.int32(0))
    npos_kept = jnp.where(condp, kept_tot, npos)
    Kn = jnp.int32(256) - npos_kept

    run_select(jnp.int32(0), Kn, gneg_hbm, negbase, nneg, rsumneg)

    pltpu.sync_copy(labelv, lab_hbm.at[pl.ds(base, NA)])


_LOGTAB = np.full((NTAB,), 0.0, np.float32)
_LOGTAB[0] = -np.inf
_LOGTAB[1:] = np.log(np.arange(1, NTAB, dtype=np.float32))


@jax.jit
def kernel(gt_bbox, anchors, img_size):
    f32 = jnp.float32
    a_pad = jnp.concatenate(
        [anchors.astype(f32), jnp.full((NT - N0, 4), -1.0, f32)], axis=0)
    a_flat = a_pad.T.reshape(-1)
    gtc = gt_bbox.astype(f32).T
    w = img_size[1].astype(f32)
    h = img_size[0].astype(f32)
    wh = jnp.concatenate([jnp.full((16,), w, f32), jnp.full((16,), h, f32)])
    key = jax.random.key(42)
    kpos, kneg = jax.random.split(key)
    gpos = jax.random.gumbel(kpos, (N0,), f32)
    gneg = jax.random.gumbel(kneg, (N0,), f32)
    pad = jnp.zeros((NTAB - N0,), f32)
    gpos = jnp.concatenate([gpos, pad])
    gneg = jnp.concatenate([gneg, pad])
    logtab = jnp.asarray(_LOGTAB)

    mesh = plsc.VectorSubcoreMesh(
        core_axis_name="c", subcore_axis_name="s", num_cores=1)
    off_flat, label = pl.kernel(
        _body,
        out_type=[jax.ShapeDtypeStruct((4 * NT,), jnp.float32),
                  jax.ShapeDtypeStruct((NT,), jnp.int32)],
        mesh=mesh,
        compiler_params=pltpu.CompilerParams(needs_layout_passes=False),
        scratch_types=[
            pltpu.VMEM((4 * NA,), jnp.float32),
            pltpu.VMEM((NA,), jnp.int32),
            pltpu.VMEM((NA,), jnp.int32),
            pltpu.VMEM((NA,), jnp.float32),
            pltpu.VMEM((NA,), jnp.int32),
            pltpu.VMEM((32 * NA,), jnp.float32),
            pltpu.VMEM((NA,), jnp.int32),
            pltpu.VMEM((NA,), jnp.int32),
            pltpu.VMEM((NA,), jnp.int32),
            pltpu.VMEM((4, 32), jnp.float32),
            pltpu.VMEM((32,), jnp.float32),
            pltpu.VMEM((32,), jnp.float32),
            pltpu.VMEM((32,), jnp.float32),
            pltpu.VMEM((16,), jnp.int32),
            pltpu.VMEM((256,), jnp.int32),
            pltpu.VMEM((NA + 8,), jnp.float32),
            pltpu.VMEM((NA + 8,), jnp.float32),
            pltpu.VMEM((4 * NA,), jnp.float32),
            pltpu.VMEM((W, 256), jnp.int32),
            pltpu.VMEM((W, 32), jnp.float32),
            pltpu.VMEM((W, 16), jnp.int32),
            pltpu.VMEM_SHARED((W, 32), jnp.float32),
            pltpu.VMEM_SHARED((W, 16), jnp.int32),
            pltpu.VMEM_SHARED((W, 256), jnp.int32),
            pltpu.VMEM_SHARED((W, 16), jnp.int32),
        ],
    )(a_flat, gtc, wh, gpos, gneg, logtab)

    offset = off_flat.reshape(4, NT)[:, :N0].T
    return offset, label[:N0]


# speedup vs baseline: 1.0094x; 1.0094x over previous
import functools
import numpy as np
import jax
import jax.numpy as jnp
from jax import lax
from jax.experimental import pallas as pl
from jax.experimental.pallas import tpu as pltpu
from jax.experimental.pallas import tpu_sc as plsc

N0 = 20000
W = 16
NA = 1280
NT = W * NA
NTAB = 24576
NCH = NA // 16
NEG_INF = np.float32(-np.inf)
IMAX = np.int32(2147483647)
IMIN = np.int32(-2147483648)


def _iota():
    return lax.iota(jnp.int32, 16)


def _sumi(vec):
    return jnp.sum(vec.astype(jnp.float32)).astype(jnp.int32)


def _ext(vec, idx):
    return _sumi(jnp.where(_iota() == idx, vec, 0))


def _ln(x):
    u = lax.bitcast_convert_type(x, jnp.int32)
    e = ((u >> 23) & 0xFF) - 127
    m = lax.bitcast_convert_type((u & 0x007FFFFF) | 0x3F800000, jnp.float32)
    big = m > jnp.float32(1.4142135)
    m = jnp.where(big, m * jnp.float32(0.5), m)
    e = (e + jnp.where(big, 1, 0)).astype(jnp.float32)
    z = (m - 1.0) / (m + 1.0)
    z2 = z * z
    p = z * (2.0 + z2 * (2.0 / 3.0 + z2 * (0.4 + z2 * (2.0 / 7.0 + z2 * (2.0 / 9.0)))))
    return e * jnp.float32(0.6931471805599453) + p


def _orderable(k):
    u = lax.bitcast_convert_type(k, jnp.int32)
    return u ^ ((u >> 31) & 0x7FFFFFFF)


def _body(a_hbm, gtc_hbm, wh_hbm, gpos_hbm, gneg_hbm, logt_hbm,
          off_hbm, lab_hbm,
          av, maskv, lrankv, pamaxv, pargv, iouv, labelv, keyv, selv,
          gtcv, whv, gmaxv, gtmaxlocv, cntv, histv, gslv, logtv, offv,
          histall, gmaxall, cntall,
          sh_gtmax, sh_cnt, sh_hist):
    wid = lax.axis_index("s")
    base = wid * NA
    iot = _iota()
    onesi = jnp.ones((16,), jnp.int32)
    zerosi = jnp.zeros((16,), jnp.int32)

    for f in range(4):
        pltpu.sync_copy(a_hbm.at[pl.ds(f * NT + base, NA)], av.at[pl.ds(f * NA, NA)])
    pltpu.sync_copy(gtc_hbm, gtcv)
    pltpu.sync_copy(wh_hbm, whv)
    wv = whv[pl.ds(0, 16)]
    hv = whv[pl.ds(16, 16)]

    def p1_chunk(c, car):
        cmask, rbase = car
        ax1 = av[pl.ds(0 * NA + c * 16, 16)]
        ay1 = av[pl.ds(1 * NA + c * 16, 16)]
        ax2 = av[pl.ds(2 * NA + c * 16, 16)]
        ay2 = av[pl.ds(3 * NA + c * 16, 16)]
        mx = (ax1 >= 0.0) & (ay1 >= 0.0) & (ax2 <= wv) & (ay2 <= hv)
        mi = jnp.where(mx, 1, 0)
        maskv[pl.ds(c * 16, 16)] = mi
        mf = jnp.where(mx, 1.0, 0.0)
        cs = (plsc.cumsum(mf) - mf).astype(jnp.int32)
        lrankv[pl.ds(c * 16, 16)] = rbase + cs
        area_a = (ax2 - ax1) * (ay2 - ay1)

        def p1_gt(j, icar):
            pamax, parg = icar
            jf = jnp.full((16,), j, jnp.int32)
            gx1 = plsc.load_gather(gtcv, [zerosi, jf])
            gy1 = plsc.load_gather(gtcv, [onesi, jf])
            gx2 = plsc.load_gather(gtcv, [onesi + onesi, jf])
            gy2 = plsc.load_gather(gtcv, [onesi + onesi + onesi, jf])
            ix1 = jnp.maximum(ax1, gx1)
            iy1 = jnp.maximum(ay1, gy1)
            ix2 = jnp.minimum(ax2, gx2)
            iy2 = jnp.minimum(ay2, gy2)
            iw = jnp.maximum(ix2 - ix1, 0.0)
            ih = jnp.maximum(iy2 - iy1, 0.0)
            inter = iw * ih
            area_b = (gx2 - gx1) * (gy2 - gy1)
            iou = jnp.where(mx, inter / (area_a + area_b - inter), NEG_INF)
            iouv[pl.ds(j * NA + c * 16, 16)] = iou
            better = iou > pamax
            pamax = jnp.where(better, iou, pamax)
            parg = jnp.where(better, jf, parg)
            return pamax, parg

        pamax0 = jnp.full((16,), NEG_INF, jnp.float32)
        pamax, parg = lax.fori_loop(0, 32, p1_gt, (pamax0, zerosi))
        pamaxv[pl.ds(c * 16, 16)] = pamax
        pargv[pl.ds(c * 16, 16)] = parg
        nm = _sumi(mi)
        return cmask + nm, rbase + nm

    ninf16 = jnp.full((16,), NEG_INF, jnp.float32)
    cmask, _ = lax.fori_loop(
        0, NCH, p1_chunk, (jnp.int32(0), jnp.int32(0)))

    def rowmax_j(j, car):
        glo, ghi = car

        def rm_c(c, acc):
            return jnp.maximum(acc, iouv[pl.ds(j * NA + c * 16, 16)])

        s = jnp.max(lax.fori_loop(0, NCH, rm_c, ninf16))
        sb = jnp.full((16,), s, jnp.float32)
        glo = jnp.where(iot == j, jnp.maximum(glo, sb), glo)
        ghi = jnp.where(iot == (j - 16), jnp.maximum(ghi, sb), ghi)
        return glo, ghi

    glo, ghi = lax.fori_loop(0, 32, rowmax_j, (ninf16, ninf16))
    gtmaxlocv[pl.ds(0, 16)] = glo
    gtmaxlocv[pl.ds(16, 16)] = ghi
    pltpu.sync_copy(gtmaxlocv, sh_gtmax.at[wid])
    plsc.subcore_barrier()

    pltpu.sync_copy(sh_gtmax, gmaxall)

    def gmax_w(w, car):
        glo, ghi = car
        return (jnp.maximum(glo, gmaxall[w, pl.ds(0, 16)]),
                jnp.maximum(ghi, gmaxall[w, pl.ds(16, 16)]))

    glo, ghi = lax.fori_loop(0, W, gmax_w, (ninf16, ninf16))
    gmaxv[pl.ds(0, 16)] = glo
    gmaxv[pl.ds(16, 16)] = ghi

    def p2_chunk(c, car):
        cpos, cneg, spos, sneg = car
        pam = pamaxv[pl.ds(c * 16, 16)]
        mi = maskv[pl.ds(c * 16, 16)]
        mx = mi != 0
        lr = lrankv[pl.ds(c * 16, 16)]

        def p2_gt(j, acc):
            gmb = plsc.load_gather(gmaxv, [jnp.full((16,), j, jnp.int32)])
            eq = (iouv[pl.ds(j * NA + c * 16, 16)] == gmb) & mx
            return acc | jnp.where(eq, 1, 0)

        isgt = lax.fori_loop(0, 32, p2_gt, zerosi) != 0
        one = isgt | (mx & (pam >= jnp.float32(0.7)))
        neg = mx & (pam <= jnp.float32(0.3)) & jnp.logical_not(one)
        lab = jnp.where(one, 1, jnp.where(neg, 0, -1))
        labelv[pl.ds(c * 16, 16)] = lab

        ax1 = av[pl.ds(0 * NA + c * 16, 16)]
        ay1 = av[pl.ds(1 * NA + c * 16, 16)]
        ax2 = av[pl.ds(2 * NA + c * 16, 16)]
        ay2 = av[pl.ds(3 * NA + c * 16, 16)]
        parg = pargv[pl.ds(c * 16, 16)]
        gx1 = plsc.load_gather(gtcv, [zerosi, parg])
        gy1 = plsc.load_gather(gtcv, [onesi, parg])
        gx2 = plsc.load_gather(gtcv, [onesi + onesi, parg])
        gy2 = plsc.load_gather(gtcv, [onesi + onesi + onesi, parg])
        aw = ax2 - ax1
        ah = ay2 - ay1
        acx = ax1 + 0.5 * aw
        acy = ay1 + 0.5 * ah
        gw = gx2 - gx1
        gh = gy2 - gy1
        gcx = gx1 + 0.5 * gw
        gcy = gy1 + 0.5 * gh
        zf = jnp.zeros((16,), jnp.float32)
        offv[pl.ds(0 * NA + c * 16, 16)] = jnp.where(mx, (gcx - acx) / aw, zf)
        offv[pl.ds(1 * NA + c * 16, 16)] = jnp.where(mx, (gcy - acy) / ah, zf)
        offv[pl.ds(2 * NA + c * 16, 16)] = jnp.where(mx, _ln(gw / aw), zf)
        offv[pl.ds(3 * NA + c * 16, 16)] = jnp.where(mx, _ln(gh / ah), zf)

        onei = jnp.where(one, 1, 0)
        negi = jnp.where(neg, 1, 0)
        return (cpos + _sumi(onei), cneg + _sumi(negi),
                spos + _sumi(jnp.where(one, lr, 0)),
                sneg + _sumi(jnp.where(neg, lr, 0)))

    cpos, cneg, spos, sneg = lax.fori_loop(
        0, NCH, p2_chunk, (jnp.int32(0),) * 4)
    for f in range(4):
        pltpu.sync_copy(offv.at[pl.ds(f * NA, NA)], off_hbm.at[pl.ds(f * NT + base, NA)])
    cv = jnp.where(iot == 0, cmask, 0)
    cv = jnp.where(iot == 1, cpos, cv)
    cv = jnp.where(iot == 2, cneg, cv)
    cv = jnp.where(iot == 3, spos, cv)
    cv = jnp.where(iot == 4, sneg, cv)
    cntv[pl.ds(0, 16)] = cv
    pltpu.sync_copy(cntv, sh_cnt.at[wid])
    plsc.subcore_barrier()

    pltpu.sync_copy(sh_cnt, cntall)

    def red_w(w, car):
        rb_all, rb_my, npos, posb, nneg, negb, rsp, rsn = car
        row = cntall[w, pl.ds(0, 16)]
        cm = _ext(row, 0)
        cp = _ext(row, 1)
        cn = _ext(row, 2)
        sp = _ext(row, 3)
        sn = _ext(row, 4)
        before = jnp.where(w < wid, 1, 0)
        return (rb_all + cm, rb_my + before * cm, npos + cp,
                posb + before * cp, nneg + cn, negb + before * cn,
                rsp + cp * rb_all + sp, rsn + cn * rb_all + sn)

    (_, rankbase, npos, posbase, nneg, negbase, rsumpos, rsumneg) = lax.fori_loop(
        0, W, red_w, (jnp.int32(0),) * 8)

    rb8 = pl.multiple_of((rankbase >> 3) << 3, 8)
    rrem = rankbase - rb8
    pltpu.sync_copy(logt_hbm.at[pl.ds(rb8, NA + 8)], logtv)

    Kp = jnp.int32(128)
    condp = (npos > Kp) & (rsumpos > 0)
    Kn = jnp.int32(256) - jnp.where(condp, Kp, npos)
    condn = (nneg > Kn) & (rsumneg > 0)
    cpi = jnp.where(condp, 1, 0)
    cni = jnp.where(condn, 1, 0)

    pb8 = pl.multiple_of((posbase >> 3) << 3, 8)
    prem = posbase - pb8
    nb8 = pl.multiple_of((negbase >> 3) << 3, 8)
    nrem = negbase - nb8
    pltpu.sync_copy(gpos_hbm.at[pl.ds(pb8, NA + 8)], gslv.at[pl.ds(0, NA + 8)])
    pltpu.sync_copy(gneg_hbm.at[pl.ds(nb8, NA + 8)], gslv.at[pl.ds(NA + 8, NA + 8)])

    def key_chunk(c, car):
        jbp, jbn = car
        lab = labelv[pl.ds(c * 16, 16)]
        lg = plsc.load_gather(logtv, [lrankv[pl.ds(c * 16, 16)] + rrem])
        selp = lab == 1
        spf = jnp.where(selp, 1.0, 0.0)
        jlp = jbp + (plsc.cumsum(spf) - spf).astype(jnp.int32)
        gvp = plsc.load_gather(gslv, [jlp + prem])
        vp = _orderable(gvp + lg)
        actp = selp & (cpi != 0)
        keyv[pl.ds(c * 16, 16)] = jnp.where(actp, vp, IMAX)
        selv[pl.ds(c * 16, 16)] = jnp.where(actp, 1, 0)
        seln = lab == 0
        snf = jnp.where(seln, 1.0, 0.0)
        jln = jbn + (plsc.cumsum(snf) - snf).astype(jnp.int32)
        gvn = plsc.load_gather(gslv, [jln + (NA + 8 + nrem)])
        vn = _orderable(gvn + lg)
        actn = seln & (cni != 0)
        keyv[pl.ds(NA + c * 16, 16)] = jnp.where(actn, vn, IMAX)
        selv[pl.ds(NA + c * 16, 16)] = jnp.where(actn, 1, 0)
        return jbp + _sumi(jnp.where(selp, 1, 0)), jbn + _sumi(jnp.where(seln, 1, 0))

    lax.fori_loop(0, NCH, key_chunk, (jnp.int32(0), jnp.int32(0)))

    def radix_round(r, pc):
        Pp, Cp, Pn, Cn = pc
        shift = 24 - 8 * r

        def zero_h(h, _):
            histv[pl.ds(h * 16, 16)] = zerosi
            return 0

        lax.fori_loop(0, 32, zero_h, 0)

        def hist_chunk(c, _):
            vp = keyv[pl.ds(c * 16, 16)]
            wp = vp ^ IMIN
            digp = lax.shift_right_logical(wp, shift) & 255
            hbp = lax.shift_right_logical(
                lax.shift_right_logical(wp, shift + 7), 1)
            actp = (selv[pl.ds(c * 16, 16)] != 0) & (hbp == Pp)
            plsc.addupdate_scatter(histv, [digp], onesi, mask=actp)
            vn = keyv[pl.ds(NA + c * 16, 16)]
            wn = vn ^ IMIN
            dign = (lax.shift_right_logical(wn, shift) & 255) + 256
            hbn = lax.shift_right_logical(
                lax.shift_right_logical(wn, shift + 7), 1)
            actn = (selv[pl.ds(NA + c * 16, 16)] != 0) & (hbn == Pn)
            plsc.addupdate_scatter(histv, [dign], onesi, mask=actn)
            return 0

        lax.fori_loop(0, NCH, hist_chunk, 0)
        pltpu.sync_copy(histv, sh_hist.at[wid])
        plsc.subcore_barrier()

        pltpu.sync_copy(sh_hist, histall)

        def merge_w(w, _):
            def merge_h(h, _):
                cur = jnp.where(w == 0, zerosi, histv[pl.ds(h * 16, 16)])
                histv[pl.ds(h * 16, 16)] = cur + histall[w, pl.ds(h * 16, 16)]
                return 0

            return lax.fori_loop(0, 32, merge_h, 0)

        lax.fori_loop(0, W, merge_w, 0)

        def find_digit(K, C, off):
            K1f = (K - C).astype(jnp.float32)

            def find_h(h, car):
                cum, found, bdig, cexcl = car
                accf = histv[pl.ds(off + h * 16, 16)].astype(jnp.float32)
                csf = plsc.cumsum(accf)
                cand = (cum.astype(jnp.float32) + csf) >= K1f
                idx = jnp.min(jnp.where(cand, iot.astype(jnp.float32), 16.0)).astype(jnp.int32)
                hit = (found == 0) & (idx < 16)
                csi = jnp.sum(jnp.where(iot == idx, csf - accf, 0.0)).astype(jnp.int32)
                bdig = jnp.where(hit, h * 16 + idx, bdig)
                cexcl = jnp.where(hit, cum + csi, cexcl)
                found = jnp.where(hit, 1, found)
                cum = cum + jnp.sum(jnp.where(iot == 15, csf, 0.0)).astype(jnp.int32)
                return cum, found, bdig, cexcl

            _, _, bdig, cexcl = lax.fori_loop(
                0, 16, find_h, (jnp.int32(0),) * 4)
            return bdig, cexcl

        bp, cep = find_digit(Kp, Cp, 0)
        bn, cen = find_digit(Kn, Cn, 256)
        plsc.subcore_barrier()
        return (Pp << 8) | bp, Cp + cep, (Pn << 8) | bn, Cn + cen

    Pp, _, Pn, _ = lax.fori_loop(
        0, 4, radix_round, (jnp.int32(0),) * 4)
    Tp = Pp ^ IMIN
    Tn = Pn ^ IMIN

    def apply_chunk(c, _):
        lab = labelv[pl.ds(c * 16, 16)]
        rmp = (selv[pl.ds(c * 16, 16)] != 0) & (keyv[pl.ds(c * 16, 16)] > Tp)
        rmn = (selv[pl.ds(NA + c * 16, 16)] != 0) & (keyv[pl.ds(NA + c * 16, 16)] > Tn)
        labelv[pl.ds(c * 16, 16)] = jnp.where(rmp | rmn, -1, lab)
        return 0

    lax.fori_loop(0, NCH, apply_chunk, 0)

    pltpu.sync_copy(labelv, lab_hbm.at[pl.ds(base, NA)])


_LOGTAB = np.full((NTAB,), 0.0, np.float32)
_LOGTAB[0] = -np.inf
_LOGTAB[1:] = np.log(np.arange(1, NTAB, dtype=np.float32))


@jax.jit
def kernel(gt_bbox, anchors, img_size):
    f32 = jnp.float32
    a_pad = jnp.concatenate(
        [anchors.astype(f32), jnp.full((NT - N0, 4), -1.0, f32)], axis=0)
    a_flat = a_pad.T.reshape(-1)
    gtc = gt_bbox.astype(f32).T
    w = img_size[1].astype(f32)
    h = img_size[0].astype(f32)
    wh = jnp.concatenate([jnp.full((16,), w, f32), jnp.full((16,), h, f32)])
    key = jax.random.key(42)
    kpos, kneg = jax.random.split(key)
    gpos = jax.random.gumbel(kpos, (N0,), f32)
    gneg = jax.random.gumbel(kneg, (N0,), f32)
    pad = jnp.zeros((NTAB - N0,), f32)
    gpos = jnp.concatenate([gpos, pad])
    gneg = jnp.concatenate([gneg, pad])
    logtab = jnp.asarray(_LOGTAB)

    mesh = plsc.VectorSubcoreMesh(
        core_axis_name="c", subcore_axis_name="s", num_cores=1)
    off_flat, label = pl.kernel(
        _body,
        out_type=[jax.ShapeDtypeStruct((4 * NT,), jnp.float32),
                  jax.ShapeDtypeStruct((NT,), jnp.int32)],
        mesh=mesh,
        compiler_params=pltpu.CompilerParams(needs_layout_passes=False),
        scratch_types=[
            pltpu.VMEM((4 * NA,), jnp.float32),
            pltpu.VMEM((NA,), jnp.int32),
            pltpu.VMEM((NA,), jnp.int32),
            pltpu.VMEM((NA,), jnp.float32),
            pltpu.VMEM((NA,), jnp.int32),
            pltpu.VMEM((32 * NA,), jnp.float32),
            pltpu.VMEM((NA,), jnp.int32),
            pltpu.VMEM((2 * NA,), jnp.int32),
            pltpu.VMEM((2 * NA,), jnp.int32),
            pltpu.VMEM((4, 32), jnp.float32),
            pltpu.VMEM((32,), jnp.float32),
            pltpu.VMEM((32,), jnp.float32),
            pltpu.VMEM((32,), jnp.float32),
            pltpu.VMEM((16,), jnp.int32),
            pltpu.VMEM((512,), jnp.int32),
            pltpu.VMEM((2 * (NA + 8),), jnp.float32),
            pltpu.VMEM((NA + 8,), jnp.float32),
            pltpu.VMEM((4 * NA,), jnp.float32),
            pltpu.VMEM((W, 512), jnp.int32),
            pltpu.VMEM((W, 32), jnp.float32),
            pltpu.VMEM((W, 16), jnp.int32),
            pltpu.VMEM_SHARED((W, 32), jnp.float32),
            pltpu.VMEM_SHARED((W, 16), jnp.int32),
            pltpu.VMEM_SHARED((W, 512), jnp.int32),
        ],
    )(a_flat, gtc, wh, gpos, gneg, logtab)

    offset = off_flat.reshape(4, NT)[:, :N0].T
    return offset, label[:N0]


# speedup vs baseline: 1.1302x; 1.1198x over previous
import functools
import numpy as np
import jax
import jax.numpy as jnp
from jax import lax
from jax.experimental import pallas as pl
from jax.experimental.pallas import tpu as pltpu
from jax.experimental.pallas import tpu_sc as plsc

N0 = 20000
W = 16
NA = 1280
NT = W * NA
NTAB = 24576
NCH = NA // 16
NEG_INF = np.float32(-np.inf)
IMAX = np.int32(2147483647)
IMIN = np.int32(-2147483648)


def _iota():
    return lax.iota(jnp.int32, 16)


def _sumi(vec):
    return jnp.sum(vec.astype(jnp.float32)).astype(jnp.int32)


def _ext(vec, idx):
    return _sumi(jnp.where(_iota() == idx, vec, 0))


def _ln(x):
    u = lax.bitcast_convert_type(x, jnp.int32)
    e = ((u >> 23) & 0xFF) - 127
    m = lax.bitcast_convert_type((u & 0x007FFFFF) | 0x3F800000, jnp.float32)
    big = m > jnp.float32(1.4142135)
    m = jnp.where(big, m * jnp.float32(0.5), m)
    e = (e + jnp.where(big, 1, 0)).astype(jnp.float32)
    z = (m - 1.0) / (m + 1.0)
    z2 = z * z
    p = z * (2.0 + z2 * (2.0 / 3.0 + z2 * (0.4 + z2 * (2.0 / 7.0 + z2 * (2.0 / 9.0)))))
    return e * jnp.float32(0.6931471805599453) + p


def _orderable(k):
    u = lax.bitcast_convert_type(k, jnp.int32)
    return u ^ ((u >> 31) & 0x7FFFFFFF)


def _body(a_hbm, gtc_hbm, wh_hbm, gpos_hbm, gneg_hbm, logt_hbm,
          off_hbm, lab_hbm,
          av, maskv, lrankv, pamaxv, pargv, iouv, labelv, keyv, selv,
          gtcv, whv, gmaxv, gtmaxlocv, cntv, histv, gslv, logtv, offv,
          histall, gmaxall, cntall,
          sh_gtmax, sh_cnt, sh_hist):
    wid = lax.axis_index("s")
    base = wid * NA
    iot = _iota()
    onesi = jnp.ones((16,), jnp.int32)
    zerosi = jnp.zeros((16,), jnp.int32)

    for f in range(4):
        pltpu.sync_copy(a_hbm.at[pl.ds(f * NT + base, NA)], av.at[pl.ds(f * NA, NA)])
    pltpu.sync_copy(gtc_hbm, gtcv)
    pltpu.sync_copy(wh_hbm, whv)
    wv = whv[pl.ds(0, 16)]
    hv = whv[pl.ds(16, 16)]

    def p1_chunk(c, car):
        cmask, rbase = car
        ax1 = av[pl.ds(0 * NA + c * 16, 16)]
        ay1 = av[pl.ds(1 * NA + c * 16, 16)]
        ax2 = av[pl.ds(2 * NA + c * 16, 16)]
        ay2 = av[pl.ds(3 * NA + c * 16, 16)]
        mx = (ax1 >= 0.0) & (ay1 >= 0.0) & (ax2 <= wv) & (ay2 <= hv)
        mi = jnp.where(mx, 1, 0)
        maskv[pl.ds(c * 16, 16)] = mi
        mf = jnp.where(mx, 1.0, 0.0)
        cs = (plsc.cumsum(mf) - mf).astype(jnp.int32)
        lrankv[pl.ds(c * 16, 16)] = rbase + cs
        area_a = (ax2 - ax1) * (ay2 - ay1)

        def p1_gt(jq, icar):
            pamax, parg = icar
            for dj in range(4):
                j = jq * 4 + dj
                jf = jnp.full((16,), j, jnp.int32)
                gx1 = plsc.load_gather(gtcv, [zerosi, jf])
                gy1 = plsc.load_gather(gtcv, [onesi, jf])
                gx2 = plsc.load_gather(gtcv, [onesi + onesi, jf])
                gy2 = plsc.load_gather(gtcv, [onesi + onesi + onesi, jf])
                ix1 = jnp.maximum(ax1, gx1)
                iy1 = jnp.maximum(ay1, gy1)
                ix2 = jnp.minimum(ax2, gx2)
                iy2 = jnp.minimum(ay2, gy2)
                iw = jnp.maximum(ix2 - ix1, 0.0)
                ih = jnp.maximum(iy2 - iy1, 0.0)
                inter = iw * ih
                area_b = (gx2 - gx1) * (gy2 - gy1)
                iou = jnp.where(mx, inter / (area_a + area_b - inter), NEG_INF)
                iouv[pl.ds(j * NA + c * 16, 16)] = iou
                better = iou > pamax
                pamax = jnp.where(better, iou, pamax)
                parg = jnp.where(better, jf, parg)
            return pamax, parg

        pamax0 = jnp.full((16,), NEG_INF, jnp.float32)
        pamax, parg = lax.fori_loop(0, 8, p1_gt, (pamax0, zerosi))
        pamaxv[pl.ds(c * 16, 16)] = pamax
        pargv[pl.ds(c * 16, 16)] = parg
        nm = _sumi(mi)
        return cmask + nm, rbase + nm

    ninf16 = jnp.full((16,), NEG_INF, jnp.float32)
    cmask, _ = lax.fori_loop(
        0, NCH, p1_chunk, (jnp.int32(0), jnp.int32(0)))

    def rowmax_j(j, car):
        glo, ghi = car

        def rm_c(cq, acc):
            a0 = iouv[pl.ds(j * NA + cq * 64, 16)]
            a1 = iouv[pl.ds(j * NA + cq * 64 + 16, 16)]
            a2 = iouv[pl.ds(j * NA + cq * 64 + 32, 16)]
            a3 = iouv[pl.ds(j * NA + cq * 64 + 48, 16)]
            return jnp.maximum(acc, jnp.maximum(jnp.maximum(a0, a1),
                                                jnp.maximum(a2, a3)))

        s = jnp.max(lax.fori_loop(0, NCH // 4, rm_c, ninf16))
        sb = jnp.full((16,), s, jnp.float32)
        glo = jnp.where(iot == j, jnp.maximum(glo, sb), glo)
        ghi = jnp.where(iot == (j - 16), jnp.maximum(ghi, sb), ghi)
        return glo, ghi

    glo, ghi = lax.fori_loop(0, 32, rowmax_j, (ninf16, ninf16))
    gtmaxlocv[pl.ds(0, 16)] = glo
    gtmaxlocv[pl.ds(16, 16)] = ghi
    pltpu.sync_copy(gtmaxlocv, sh_gtmax.at[wid])
    plsc.subcore_barrier()

    pltpu.sync_copy(sh_gtmax, gmaxall)

    def gmax_w(w, car):
        glo, ghi = car
        return (jnp.maximum(glo, gmaxall[w, pl.ds(0, 16)]),
                jnp.maximum(ghi, gmaxall[w, pl.ds(16, 16)]))

    glo, ghi = lax.fori_loop(0, W, gmax_w, (ninf16, ninf16))
    gmaxv[pl.ds(0, 16)] = glo
    gmaxv[pl.ds(16, 16)] = ghi

    def p2_chunk(c, car):
        cpos, cneg, spos, sneg = car
        pam = pamaxv[pl.ds(c * 16, 16)]
        mi = maskv[pl.ds(c * 16, 16)]
        mx = mi != 0
        lr = lrankv[pl.ds(c * 16, 16)]

        def p2_gt(jq, acc):
            e = []
            for dj in range(4):
                j = jq * 4 + dj
                gmb = plsc.load_gather(gmaxv, [jnp.full((16,), j, jnp.int32)])
                eq = (iouv[pl.ds(j * NA + c * 16, 16)] == gmb) & mx
                e.append(jnp.where(eq, 1, 0))
            return acc | (e[0] | e[1]) | (e[2] | e[3])

        isgt = lax.fori_loop(0, 8, p2_gt, zerosi) != 0
        one = isgt | (mx & (pam >= jnp.float32(0.7)))
        neg = mx & (pam <= jnp.float32(0.3)) & jnp.logical_not(one)
        lab = jnp.where(one, 1, jnp.where(neg, 0, -1))
        labelv[pl.ds(c * 16, 16)] = lab

        ax1 = av[pl.ds(0 * NA + c * 16, 16)]
        ay1 = av[pl.ds(1 * NA + c * 16, 16)]
        ax2 = av[pl.ds(2 * NA + c * 16, 16)]
        ay2 = av[pl.ds(3 * NA + c * 16, 16)]
        parg = pargv[pl.ds(c * 16, 16)]
        gx1 = plsc.load_gather(gtcv, [zerosi, parg])
        gy1 = plsc.load_gather(gtcv, [onesi, parg])
        gx2 = plsc.load_gather(gtcv, [onesi + onesi, parg])
        gy2 = plsc.load_gather(gtcv, [onesi + onesi + onesi, parg])
        aw = ax2 - ax1
        ah = ay2 - ay1
        acx = ax1 + 0.5 * aw
        acy = ay1 + 0.5 * ah
        gw = gx2 - gx1
        gh = gy2 - gy1
        gcx = gx1 + 0.5 * gw
        gcy = gy1 + 0.5 * gh
        zf = jnp.zeros((16,), jnp.float32)
        offv[pl.ds(0 * NA + c * 16, 16)] = jnp.where(mx, (gcx - acx) / aw, zf)
        offv[pl.ds(1 * NA + c * 16, 16)] = jnp.where(mx, (gcy - acy) / ah, zf)
        offv[pl.ds(2 * NA + c * 16, 16)] = jnp.where(mx, _ln(gw / aw), zf)
        offv[pl.ds(3 * NA + c * 16, 16)] = jnp.where(mx, _ln(gh / ah), zf)

        onei = jnp.where(one, 1, 0)
        negi = jnp.where(neg, 1, 0)
        return (cpos + _sumi(onei), cneg + _sumi(negi),
                spos + _sumi(jnp.where(one, lr, 0)),
                sneg + _sumi(jnp.where(neg, lr, 0)))

    cpos, cneg, spos, sneg = lax.fori_loop(
        0, NCH, p2_chunk, (jnp.int32(0),) * 4)
    for f in range(4):
        pltpu.sync_copy(offv.at[pl.ds(f * NA, NA)], off_hbm.at[pl.ds(f * NT + base, NA)])
    cv = jnp.where(iot == 0, cmask, 0)
    cv = jnp.where(iot == 1, cpos, cv)
    cv = jnp.where(iot == 2, cneg, cv)
    cv = jnp.where(iot == 3, spos, cv)
    cv = jnp.where(iot == 4, sneg, cv)
    cntv[pl.ds(0, 16)] = cv
    pltpu.sync_copy(cntv, sh_cnt.at[wid])
    plsc.subcore_barrier()

    pltpu.sync_copy(sh_cnt, cntall)

    def red_w(w, car):
        rb_all, rb_my, npos, posb, nneg, negb, rsp, rsn = car
        row = cntall[w, pl.ds(0, 16)]
        cm = _ext(row, 0)
        cp = _ext(row, 1)
        cn = _ext(row, 2)
        sp = _ext(row, 3)
        sn = _ext(row, 4)
        before = jnp.where(w < wid, 1, 0)
        return (rb_all + cm, rb_my + before * cm, npos + cp,
                posb + before * cp, nneg + cn, negb + before * cn,
                rsp + cp * rb_all + sp, rsn + cn * rb_all + sn)

    (_, rankbase, npos, posbase, nneg, negbase, rsumpos, rsumneg) = lax.fori_loop(
        0, W, red_w, (jnp.int32(0),) * 8)

    rb8 = pl.multiple_of((rankbase >> 3) << 3, 8)
    rrem = rankbase - rb8
    pltpu.sync_copy(logt_hbm.at[pl.ds(rb8, NA + 8)], logtv)

    Kp = jnp.int32(128)
    condp = (npos > Kp) & (rsumpos > 0)
    Kn = jnp.int32(256) - jnp.where(condp, Kp, npos)
    condn = (nneg > Kn) & (rsumneg > 0)
    cpi = jnp.where(condp, 1, 0)
    cni = jnp.where(condn, 1, 0)

    pb8 = pl.multiple_of((posbase >> 3) << 3, 8)
    prem = posbase - pb8
    nb8 = pl.multiple_of((negbase >> 3) << 3, 8)
    nrem = negbase - nb8
    pltpu.sync_copy(gpos_hbm.at[pl.ds(pb8, NA + 8)], gslv.at[pl.ds(0, NA + 8)])
    pltpu.sync_copy(gneg_hbm.at[pl.ds(nb8, NA + 8)], gslv.at[pl.ds(NA + 8, NA + 8)])

    def key_chunk(c, car):
        jbp, jbn = car
        lab = labelv[pl.ds(c * 16, 16)]
        lg = plsc.load_gather(logtv, [lrankv[pl.ds(c * 16, 16)] + rrem])
        selp = lab == 1
        spf = jnp.where(selp, 1.0, 0.0)
        jlp = jbp + (plsc.cumsum(spf) - spf).astype(jnp.int32)
        gvp = plsc.load_gather(gslv, [jlp + prem])
        vp = _orderable(gvp + lg)
        actp = selp & (cpi != 0)
        keyv[pl.ds(c * 16, 16)] = jnp.where(actp, vp, IMAX)
        selv[pl.ds(c * 16, 16)] = jnp.where(actp, 1, 0)
        seln = lab == 0
        snf = jnp.where(seln, 1.0, 0.0)
        jln = jbn + (plsc.cumsum(snf) - snf).astype(jnp.int32)
        gvn = plsc.load_gather(gslv, [jln + (NA + 8 + nrem)])
        vn = _orderable(gvn + lg)
        actn = seln & (cni != 0)
        keyv[pl.ds(NA + c * 16, 16)] = jnp.where(actn, vn, IMAX)
        selv[pl.ds(NA + c * 16, 16)] = jnp.where(actn, 1, 0)
        return jbp + _sumi(jnp.where(selp, 1, 0)), jbn + _sumi(jnp.where(seln, 1, 0))

    lax.fori_loop(0, NCH, key_chunk, (jnp.int32(0), jnp.int32(0)))

    def radix_round(r, pc):
        Pp, Cp, Pn, Cn = pc
        shift = 24 - 8 * r

        def zero_h(h, _):
            histv[pl.ds(h * 16, 16)] = zerosi
            return 0

        lax.fori_loop(0, 32, zero_h, 0)

        def hist_chunk(c, _):
            vp = keyv[pl.ds(c * 16, 16)]
            wp = vp ^ IMIN
            digp = lax.shift_right_logical(wp, shift) & 255
            hbp = lax.shift_right_logical(
                lax.shift_right_logical(wp, shift + 7), 1)
            actp = (selv[pl.ds(c * 16, 16)] != 0) & (hbp == Pp)
            plsc.addupdate_scatter(histv, [digp], onesi, mask=actp)
            vn = keyv[pl.ds(NA + c * 16, 16)]
            wn = vn ^ IMIN
            dign = (lax.shift_right_logical(wn, shift) & 255) + 256
            hbn = lax.shift_right_logical(
                lax.shift_right_logical(wn, shift + 7), 1)
            actn = (selv[pl.ds(NA + c * 16, 16)] != 0) & (hbn == Pn)
            plsc.addupdate_scatter(histv, [dign], onesi, mask=actn)
            return 0

        lax.fori_loop(0, NCH, hist_chunk, 0)
        pltpu.sync_copy(histv, sh_hist.at[wid])
        plsc.subcore_barrier()

        pltpu.sync_copy(sh_hist, histall)

        def merge_w(w, _):
            def merge_h(h, _):
                cur = jnp.where(w == 0, zerosi, histv[pl.ds(h * 16, 16)])
                histv[pl.ds(h * 16, 16)] = cur + histall[w, pl.ds(h * 16, 16)]
                return 0

            return lax.fori_loop(0, 32, merge_h, 0)

        lax.fori_loop(0, W, merge_w, 0)

        def find_digit(K, C, off):
            K1f = (K - C).astype(jnp.float32)

            def find_h(h, car):
                cum, found, bdig, cexcl = car
                accf = histv[pl.ds(off + h * 16, 16)].astype(jnp.float32)
                csf = plsc.cumsum(accf)
                cand = (cum.astype(jnp.float32) + csf) >= K1f
                idx = jnp.min(jnp.where(cand, iot.astype(jnp.float32), 16.0)).astype(jnp.int32)
                hit = (found == 0) & (idx < 16)
                csi = jnp.sum(jnp.where(iot == idx, csf - accf, 0.0)).astype(jnp.int32)
                bdig = jnp.where(hit, h * 16 + idx, bdig)
                cexcl = jnp.where(hit, cum + csi, cexcl)
                found = jnp.where(hit, 1, found)
                cum = cum + jnp.sum(jnp.where(iot == 15, csf, 0.0)).astype(jnp.int32)
                return cum, found, bdig, cexcl

            _, _, bdig, cexcl = lax.fori_loop(
                0, 16, find_h, (jnp.int32(0),) * 4)
            return bdig, cexcl

        bp, cep = find_digit(Kp, Cp, 0)
        bn, cen = find_digit(Kn, Cn, 256)
        plsc.subcore_barrier()
        return (Pp << 8) | bp, Cp + cep, (Pn << 8) | bn, Cn + cen

    Pp, _, Pn, _ = lax.fori_loop(
        0, 4, radix_round, (jnp.int32(0),) * 4)
    Tp = Pp ^ IMIN
    Tn = Pn ^ IMIN

    def apply_chunk(c, _):
        lab = labelv[pl.ds(c * 16, 16)]
        rmp = (selv[pl.ds(c * 16, 16)] != 0) & (keyv[pl.ds(c * 16, 16)] > Tp)
        rmn = (selv[pl.ds(NA + c * 16, 16)] != 0) & (keyv[pl.ds(NA + c * 16, 16)] > Tn)
        labelv[pl.ds(c * 16, 16)] = jnp.where(rmp | rmn, -1, lab)
        return 0

    lax.fori_loop(0, NCH, apply_chunk, 0)

    pltpu.sync_copy(labelv, lab_hbm.at[pl.ds(base, NA)])


_LOGTAB = np.full((NTAB,), 0.0, np.float32)
_LOGTAB[0] = -np.inf
_LOGTAB[1:] = np.log(np.arange(1, NTAB, dtype=np.float32))


@jax.jit
def kernel(gt_bbox, anchors, img_size):
    f32 = jnp.float32
    a_pad = jnp.concatenate(
        [anchors.astype(f32), jnp.full((NT - N0, 4), -1.0, f32)], axis=0)
    a_flat = a_pad.T.reshape(-1)
    gtc = gt_bbox.astype(f32).T
    w = img_size[1].astype(f32)
    h = img_size[0].astype(f32)
    wh = jnp.concatenate([jnp.full((16,), w, f32), jnp.full((16,), h, f32)])
    key = jax.random.key(42)
    kpos, kneg = jax.random.split(key)
    gpos = jax.random.gumbel(kpos, (N0,), f32)
    gneg = jax.random.gumbel(kneg, (N0,), f32)
    pad = jnp.zeros((NTAB - N0,), f32)
    gpos = jnp.concatenate([gpos, pad])
    gneg = jnp.concatenate([gneg, pad])
    logtab = jnp.asarray(_LOGTAB)

    mesh = plsc.VectorSubcoreMesh(
        core_axis_name="c", subcore_axis_name="s", num_cores=1)
    off_flat, label = pl.kernel(
        _body,
        out_type=[jax.ShapeDtypeStruct((4 * NT,), jnp.float32),
                  jax.ShapeDtypeStruct((NT,), jnp.int32)],
        mesh=mesh,
        compiler_params=pltpu.CompilerParams(needs_layout_passes=False),
        scratch_types=[
            pltpu.VMEM((4 * NA,), jnp.float32),
            pltpu.VMEM((NA,), jnp.int32),
            pltpu.VMEM((NA,), jnp.int32),
            pltpu.VMEM((NA,), jnp.float32),
            pltpu.VMEM((NA,), jnp.int32),
            pltpu.VMEM((32 * NA,), jnp.float32),
            pltpu.VMEM((NA,), jnp.int32),
            pltpu.VMEM((2 * NA,), jnp.int32),
            pltpu.VMEM((2 * NA,), jnp.int32),
            pltpu.VMEM((4, 32), jnp.float32),
            pltpu.VMEM((32,), jnp.float32),
            pltpu.VMEM((32,), jnp.float32),
            pltpu.VMEM((32,), jnp.float32),
            pltpu.VMEM((16,), jnp.int32),
            pltpu.VMEM((512,), jnp.int32),
            pltpu.VMEM((2 * (NA + 8),), jnp.float32),
            pltpu.VMEM((NA + 8,), jnp.float32),
            pltpu.VMEM((4 * NA,), jnp.float32),
            pltpu.VMEM((W, 512), jnp.int32),
            pltpu.VMEM((W, 32), jnp.float32),
            pltpu.VMEM((W, 16), jnp.int32),
            pltpu.VMEM_SHARED((W, 32), jnp.float32),
            pltpu.VMEM_SHARED((W, 16), jnp.int32),
            pltpu.VMEM_SHARED((W, 512), jnp.int32),
        ],
    )(a_flat, gtc, wh, gpos, gneg, logtab)

    offset = off_flat.reshape(4, NT)[:, :N0].T
    return offset, label[:N0]


# speedup vs baseline: 1.1344x; 1.0037x over previous
import functools
import numpy as np
import jax
import jax.numpy as jnp
from jax import lax
from jax.experimental import pallas as pl
from jax.experimental.pallas import tpu as pltpu
from jax.experimental.pallas import tpu_sc as plsc

N0 = 20000
W = 16
NA = 1280
NT = W * NA
NTAB = 24576
NCH = NA // 16
NEG_INF = np.float32(-np.inf)
IMAX = np.int32(2147483647)
IMIN = np.int32(-2147483648)


def _iota():
    return lax.iota(jnp.int32, 16)


def _sumi(vec):
    return jnp.sum(vec.astype(jnp.float32)).astype(jnp.int32)


def _ext(vec, idx):
    return _sumi(jnp.where(_iota() == idx, vec, 0))


def _ln(x):
    u = lax.bitcast_convert_type(x, jnp.int32)
    e = ((u >> 23) & 0xFF) - 127
    m = lax.bitcast_convert_type((u & 0x007FFFFF) | 0x3F800000, jnp.float32)
    big = m > jnp.float32(1.4142135)
    m = jnp.where(big, m * jnp.float32(0.5), m)
    e = (e + jnp.where(big, 1, 0)).astype(jnp.float32)
    z = (m - 1.0) / (m + 1.0)
    z2 = z * z
    p = z * (2.0 + z2 * (2.0 / 3.0 + z2 * (0.4 + z2 * (2.0 / 7.0 + z2 * (2.0 / 9.0)))))
    return e * jnp.float32(0.6931471805599453) + p


def _orderable(k):
    u = lax.bitcast_convert_type(k, jnp.int32)
    return u ^ ((u >> 31) & 0x7FFFFFFF)


def _body(a_hbm, gtc_hbm, wh_hbm, gpos_hbm, gneg_hbm, logt_hbm,
          off_hbm, lab_hbm,
          av, maskv, lrankv, pamaxv, pargv, iouv, labelv, keyv, selv,
          gtcv, whv, gmaxv, gtmaxlocv, cntv, histv, gslv, logtv, offv,
          histall, gmaxall, cntall,
          sh_gtmax, sh_cnt, sh_hist):
    wid = lax.axis_index("s")
    base = wid * NA
    iot = _iota()
    onesi = jnp.ones((16,), jnp.int32)
    zerosi = jnp.zeros((16,), jnp.int32)

    for f in range(4):
        pltpu.sync_copy(a_hbm.at[pl.ds(f * NT + base, NA)], av.at[pl.ds(f * NA, NA)])
    pltpu.sync_copy(gtc_hbm, gtcv)
    pltpu.sync_copy(wh_hbm, whv)
    wv = whv[pl.ds(0, 16)]
    hv = whv[pl.ds(16, 16)]

    def p1_chunk(c, car):
        cmask, rbase = car
        ax1 = av[pl.ds(0 * NA + c * 16, 16)]
        ay1 = av[pl.ds(1 * NA + c * 16, 16)]
        ax2 = av[pl.ds(2 * NA + c * 16, 16)]
        ay2 = av[pl.ds(3 * NA + c * 16, 16)]
        mx = (ax1 >= 0.0) & (ay1 >= 0.0) & (ax2 <= wv) & (ay2 <= hv)
        mi = jnp.where(mx, 1, 0)
        maskv[pl.ds(c * 16, 16)] = mi
        mf = jnp.where(mx, 1.0, 0.0)
        cs = (plsc.cumsum(mf) - mf).astype(jnp.int32)
        lrankv[pl.ds(c * 16, 16)] = rbase + cs
        area_a = (ax2 - ax1) * (ay2 - ay1)

        def p1_gt(jq, icar):
            pamax, parg = icar
            for dj in range(8):
                j = jq * 8 + dj
                jf = jnp.full((16,), j, jnp.int32)
                gx1 = plsc.load_gather(gtcv, [zerosi, jf])
                gy1 = plsc.load_gather(gtcv, [onesi, jf])
                gx2 = plsc.load_gather(gtcv, [onesi + onesi, jf])
                gy2 = plsc.load_gather(gtcv, [onesi + onesi + onesi, jf])
                ix1 = jnp.maximum(ax1, gx1)
                iy1 = jnp.maximum(ay1, gy1)
                ix2 = jnp.minimum(ax2, gx2)
                iy2 = jnp.minimum(ay2, gy2)
                iw = jnp.maximum(ix2 - ix1, 0.0)
                ih = jnp.maximum(iy2 - iy1, 0.0)
                inter = iw * ih
                area_b = (gx2 - gx1) * (gy2 - gy1)
                iou = jnp.where(mx, inter / (area_a + area_b - inter), NEG_INF)
                iouv[pl.ds(j * NA + c * 16, 16)] = iou
                better = iou > pamax
                pamax = jnp.where(better, iou, pamax)
                parg = jnp.where(better, jf, parg)
            return pamax, parg

        pamax0 = jnp.full((16,), NEG_INF, jnp.float32)
        pamax, parg = lax.fori_loop(0, 4, p1_gt, (pamax0, zerosi))
        pamaxv[pl.ds(c * 16, 16)] = pamax
        pargv[pl.ds(c * 16, 16)] = parg
        nm = _sumi(mi)
        return cmask + nm, rbase + nm

    ninf16 = jnp.full((16,), NEG_INF, jnp.float32)
    cmask, _ = lax.fori_loop(
        0, NCH, p1_chunk, (jnp.int32(0), jnp.int32(0)))

    def rowmax_j(j, car):
        glo, ghi = car

        def rm_c(cq, acc):
            a0 = iouv[pl.ds(j * NA + cq * 64, 16)]
            a1 = iouv[pl.ds(j * NA + cq * 64 + 16, 16)]
            a2 = iouv[pl.ds(j * NA + cq * 64 + 32, 16)]
            a3 = iouv[pl.ds(j * NA + cq * 64 + 48, 16)]
            return jnp.maximum(acc, jnp.maximum(jnp.maximum(a0, a1),
                                                jnp.maximum(a2, a3)))

        s = jnp.max(lax.fori_loop(0, NCH // 4, rm_c, ninf16))
        sb = jnp.full((16,), s, jnp.float32)
        glo = jnp.where(iot == j, jnp.maximum(glo, sb), glo)
        ghi = jnp.where(iot == (j - 16), jnp.maximum(ghi, sb), ghi)
        return glo, ghi

    glo, ghi = lax.fori_loop(0, 32, rowmax_j, (ninf16, ninf16))
    gtmaxlocv[pl.ds(0, 16)] = glo
    gtmaxlocv[pl.ds(16, 16)] = ghi
    pltpu.sync_copy(gtmaxlocv, sh_gtmax.at[wid])
    plsc.subcore_barrier()

    pltpu.sync_copy(sh_gtmax, gmaxall)

    def gmax_w(w, car):
        glo, ghi = car
        return (jnp.maximum(glo, gmaxall[w, pl.ds(0, 16)]),
                jnp.maximum(ghi, gmaxall[w, pl.ds(16, 16)]))

    glo, ghi = lax.fori_loop(0, W, gmax_w, (ninf16, ninf16))
    gmaxv[pl.ds(0, 16)] = glo
    gmaxv[pl.ds(16, 16)] = ghi

    def p2_chunk(c, car):
        cpos, cneg, spos, sneg = car
        pam = pamaxv[pl.ds(c * 16, 16)]
        mi = maskv[pl.ds(c * 16, 16)]
        mx = mi != 0
        lr = lrankv[pl.ds(c * 16, 16)]

        def p2_gt(jq, acc):
            e = []
            for dj in range(8):
                j = jq * 8 + dj
                gmb = plsc.load_gather(gmaxv, [jnp.full((16,), j, jnp.int32)])
                eq = (iouv[pl.ds(j * NA + c * 16, 16)] == gmb) & mx
                e.append(jnp.where(eq, 1, 0))
            return acc | ((e[0] | e[1]) | (e[2] | e[3])) | ((e[4] | e[5]) | (e[6] | e[7]))

        isgt = lax.fori_loop(0, 4, p2_gt, zerosi) != 0
        one = isgt | (mx & (pam >= jnp.float32(0.7)))
        neg = mx & (pam <= jnp.float32(0.3)) & jnp.logical_not(one)
        lab = jnp.where(one, 1, jnp.where(neg, 0, -1))
        labelv[pl.ds(c * 16, 16)] = lab

        ax1 = av[pl.ds(0 * NA + c * 16, 16)]
        ay1 = av[pl.ds(1 * NA + c * 16, 16)]
        ax2 = av[pl.ds(2 * NA + c * 16, 16)]
        ay2 = av[pl.ds(3 * NA + c * 16, 16)]
        parg = pargv[pl.ds(c * 16, 16)]
        gx1 = plsc.load_gather(gtcv, [zerosi, parg])
        gy1 = plsc.load_gather(gtcv, [onesi, parg])
        gx2 = plsc.load_gather(gtcv, [onesi + onesi, parg])
        gy2 = plsc.load_gather(gtcv, [onesi + onesi + onesi, parg])
        aw = ax2 - ax1
        ah = ay2 - ay1
        acx = ax1 + 0.5 * aw
        acy = ay1 + 0.5 * ah
        gw = gx2 - gx1
        gh = gy2 - gy1
        gcx = gx1 + 0.5 * gw
        gcy = gy1 + 0.5 * gh
        zf = jnp.zeros((16,), jnp.float32)
        offv[pl.ds(0 * NA + c * 16, 16)] = jnp.where(mx, (gcx - acx) / aw, zf)
        offv[pl.ds(1 * NA + c * 16, 16)] = jnp.where(mx, (gcy - acy) / ah, zf)
        offv[pl.ds(2 * NA + c * 16, 16)] = jnp.where(mx, _ln(gw / aw), zf)
        offv[pl.ds(3 * NA + c * 16, 16)] = jnp.where(mx, _ln(gh / ah), zf)

        onei = jnp.where(one, 1, 0)
        negi = jnp.where(neg, 1, 0)
        return (cpos + _sumi(onei), cneg + _sumi(negi),
                spos + _sumi(jnp.where(one, lr, 0)),
                sneg + _sumi(jnp.where(neg, lr, 0)))

    cpos, cneg, spos, sneg = lax.fori_loop(
        0, NCH, p2_chunk, (jnp.int32(0),) * 4)
    for f in range(4):
        pltpu.sync_copy(offv.at[pl.ds(f * NA, NA)], off_hbm.at[pl.ds(f * NT + base, NA)])
    cv = jnp.where(iot == 0, cmask, 0)
    cv = jnp.where(iot == 1, cpos, cv)
    cv = jnp.where(iot == 2, cneg, cv)
    cv = jnp.where(iot == 3, spos, cv)
    cv = jnp.where(iot == 4, sneg, cv)
    cntv[pl.ds(0, 16)] = cv
    pltpu.sync_copy(cntv, sh_cnt.at[wid])
    plsc.subcore_barrier()

    pltpu.sync_copy(sh_cnt, cntall)

    def red_w(w, car):
        rb_all, rb_my, npos, posb, nneg, negb, rsp, rsn = car
        row = cntall[w, pl.ds(0, 16)]
        cm = _ext(row, 0)
        cp = _ext(row, 1)
        cn = _ext(row, 2)
        sp = _ext(row, 3)
        sn = _ext(row, 4)
        before = jnp.where(w < wid, 1, 0)
        return (rb_all + cm, rb_my + before * cm, npos + cp,
                posb + before * cp, nneg + cn, negb + before * cn,
                rsp + cp * rb_all + sp, rsn + cn * rb_all + sn)

    (_, rankbase, npos, posbase, nneg, negbase, rsumpos, rsumneg) = lax.fori_loop(
        0, W, red_w, (jnp.int32(0),) * 8)

    rb8 = pl.multiple_of((rankbase >> 3) << 3, 8)
    rrem = rankbase - rb8
    pltpu.sync_copy(logt_hbm.at[pl.ds(rb8, NA + 8)], logtv)

    Kp = jnp.int32(128)
    condp = (npos > Kp) & (rsumpos > 0)
    Kn = jnp.int32(256) - jnp.where(condp, Kp, npos)
    condn = (nneg > Kn) & (rsumneg > 0)
    cpi = jnp.where(condp, 1, 0)
    cni = jnp.where(condn, 1, 0)

    pb8 = pl.multiple_of((posbase >> 3) << 3, 8)
    prem = posbase - pb8
    nb8 = pl.multiple_of((negbase >> 3) << 3, 8)
    nrem = negbase - nb8
    pltpu.sync_copy(gpos_hbm.at[pl.ds(pb8, NA + 8)], gslv.at[pl.ds(0, NA + 8)])
    pltpu.sync_copy(gneg_hbm.at[pl.ds(nb8, NA + 8)], gslv.at[pl.ds(NA + 8, NA + 8)])

    def key_chunk(c, car):
        jbp, jbn = car
        lab = labelv[pl.ds(c * 16, 16)]
        lg = plsc.load_gather(logtv, [lrankv[pl.ds(c * 16, 16)] + rrem])
        selp = lab == 1
        spf = jnp.where(selp, 1.0, 0.0)
        jlp = jbp + (plsc.cumsum(spf) - spf).astype(jnp.int32)
        gvp = plsc.load_gather(gslv, [jlp + prem])
        vp = _orderable(gvp + lg)
        actp = selp & (cpi != 0)
        keyv[pl.ds(c * 16, 16)] = jnp.where(actp, vp, IMAX)
        selv[pl.ds(c * 16, 16)] = jnp.where(actp, 1, 0)
        seln = lab == 0
        snf = jnp.where(seln, 1.0, 0.0)
        jln = jbn + (plsc.cumsum(snf) - snf).astype(jnp.int32)
        gvn = plsc.load_gather(gslv, [jln + (NA + 8 + nrem)])
        vn = _orderable(gvn + lg)
        actn = seln & (cni != 0)
        keyv[pl.ds(NA + c * 16, 16)] = jnp.where(actn, vn, IMAX)
        selv[pl.ds(NA + c * 16, 16)] = jnp.where(actn, 1, 0)
        return jbp + _sumi(jnp.where(selp, 1, 0)), jbn + _sumi(jnp.where(seln, 1, 0))

    lax.fori_loop(0, NCH, key_chunk, (jnp.int32(0), jnp.int32(0)))

    def radix_round(r, pc):
        Pp, Cp, Pn, Cn = pc
        shift = 24 - 8 * r

        def zero_h(h, _):
            histv[pl.ds(h * 16, 16)] = zerosi
            return 0

        lax.fori_loop(0, 32, zero_h, 0)

        def hist_chunk(c, _):
            vp = keyv[pl.ds(c * 16, 16)]
            wp = vp ^ IMIN
            digp = lax.shift_right_logical(wp, shift) & 255
            hbp = lax.shift_right_logical(
                lax.shift_right_logical(wp, shift + 7), 1)
            actp = (selv[pl.ds(c * 16, 16)] != 0) & (hbp == Pp)
            plsc.addupdate_scatter(histv, [digp], onesi, mask=actp)
            vn = keyv[pl.ds(NA + c * 16, 16)]
            wn = vn ^ IMIN
            dign = (lax.shift_right_logical(wn, shift) & 255) + 256
            hbn = lax.shift_right_logical(
                lax.shift_right_logical(wn, shift + 7), 1)
            actn = (selv[pl.ds(NA + c * 16, 16)] != 0) & (hbn == Pn)
            plsc.addupdate_scatter(histv, [dign], onesi, mask=actn)
            return 0

        lax.fori_loop(0, NCH, hist_chunk, 0)
        pltpu.sync_copy(histv, sh_hist.at[wid])
        plsc.subcore_barrier()

        pltpu.sync_copy(sh_hist, histall)

        def merge_w(w, _):
            def merge_h(h, _):
                cur = jnp.where(w == 0, zerosi, histv[pl.ds(h * 16, 16)])
                histv[pl.ds(h * 16, 16)] = cur + histall[w, pl.ds(h * 16, 16)]
                return 0

            return lax.fori_loop(0, 32, merge_h, 0)

        lax.fori_loop(0, W, merge_w, 0)

        def find_digit(K, C, off):
            K1f = (K - C).astype(jnp.float32)

            def find_h(h, car):
                cum, found, bdig, cexcl = car
                accf = histv[pl.ds(off + h * 16, 16)].astype(jnp.float32)
                csf = plsc.cumsum(accf)
                cand = (cum.astype(jnp.float32) + csf) >= K1f
                idx = jnp.min(jnp.where(cand, iot.astype(jnp.float32), 16.0)).astype(jnp.int32)
                hit = (found == 0) & (idx < 16)
                csi = jnp.sum(jnp.where(iot == idx, csf - accf, 0.0)).astype(jnp.int32)
                bdig = jnp.where(hit, h * 16 + idx, bdig)
                cexcl = jnp.where(hit, cum + csi, cexcl)
                found = jnp.where(hit, 1, found)
                cum = cum + jnp.sum(jnp.where(iot == 15, csf, 0.0)).astype(jnp.int32)
                return cum, found, bdig, cexcl

            _, _, bdig, cexcl = lax.fori_loop(
                0, 16, find_h, (jnp.int32(0),) * 4)
            return bdig, cexcl

        bp, cep = find_digit(Kp, Cp, 0)
        bn, cen = find_digit(Kn, Cn, 256)
        plsc.subcore_barrier()
        return (Pp << 8) | bp, Cp + cep, (Pn << 8) | bn, Cn + cen

    Pp, _, Pn, _ = lax.fori_loop(
        0, 4, radix_round, (jnp.int32(0),) * 4)
    Tp = Pp ^ IMIN
    Tn = Pn ^ IMIN

    def apply_chunk(c, _):
        lab = labelv[pl.ds(c * 16, 16)]
        rmp = (selv[pl.ds(c * 16, 16)] != 0) & (keyv[pl.ds(c * 16, 16)] > Tp)
        rmn = (selv[pl.ds(NA + c * 16, 16)] != 0) & (keyv[pl.ds(NA + c * 16, 16)] > Tn)
        labelv[pl.ds(c * 16, 16)] = jnp.where(rmp | rmn, -1, lab)
        return 0

    lax.fori_loop(0, NCH, apply_chunk, 0)

    pltpu.sync_copy(labelv, lab_hbm.at[pl.ds(base, NA)])


_LOGTAB = np.full((NTAB,), 0.0, np.float32)
_LOGTAB[0] = -np.inf
_LOGTAB[1:] = np.log(np.arange(1, NTAB, dtype=np.float32))


@jax.jit
def kernel(gt_bbox, anchors, img_size):
    f32 = jnp.float32
    a_pad = jnp.concatenate(
        [anchors.astype(f32), jnp.full((NT - N0, 4), -1.0, f32)], axis=0)
    a_flat = a_pad.T.reshape(-1)
    gtc = gt_bbox.astype(f32).T
    w = img_size[1].astype(f32)
    h = img_size[0].astype(f32)
    wh = jnp.concatenate([jnp.full((16,), w, f32), jnp.full((16,), h, f32)])
    key = jax.random.key(42)
    kpos, kneg = jax.random.split(key)
    gpos = jax.random.gumbel(kpos, (N0,), f32)
    gneg = jax.random.gumbel(kneg, (N0,), f32)
    pad = jnp.zeros((NTAB - N0,), f32)
    gpos = jnp.concatenate([gpos, pad])
    gneg = jnp.concatenate([gneg, pad])
    logtab = jnp.asarray(_LOGTAB)

    mesh = plsc.VectorSubcoreMesh(
        core_axis_name="c", subcore_axis_name="s", num_cores=1)
    off_flat, label = pl.kernel(
        _body,
        out_type=[jax.ShapeDtypeStruct((4 * NT,), jnp.float32),
                  jax.ShapeDtypeStruct((NT,), jnp.int32)],
        mesh=mesh,
        compiler_params=pltpu.CompilerParams(needs_layout_passes=False),
        scratch_types=[
            pltpu.VMEM((4 * NA,), jnp.float32),
            pltpu.VMEM((NA,), jnp.int32),
            pltpu.VMEM((NA,), jnp.int32),
            pltpu.VMEM((NA,), jnp.float32),
            pltpu.VMEM((NA,), jnp.int32),
            pltpu.VMEM((32 * NA,), jnp.float32),
            pltpu.VMEM((NA,), jnp.int32),
            pltpu.VMEM((2 * NA,), jnp.int32),
            pltpu.VMEM((2 * NA,), jnp.int32),
            pltpu.VMEM((4, 32), jnp.float32),
            pltpu.VMEM((32,), jnp.float32),
            pltpu.VMEM((32,), jnp.float32),
            pltpu.VMEM((32,), jnp.float32),
            pltpu.VMEM((16,), jnp.int32),
            pltpu.VMEM((512,), jnp.int32),
            pltpu.VMEM((2 * (NA + 8),), jnp.float32),
            pltpu.VMEM((NA + 8,), jnp.float32),
            pltpu.VMEM((4 * NA,), jnp.float32),
            pltpu.VMEM((W, 512), jnp.int32),
            pltpu.VMEM((W, 32), jnp.float32),
            pltpu.VMEM((W, 16), jnp.int32),
            pltpu.VMEM_SHARED((W, 32), jnp.float32),
            pltpu.VMEM_SHARED((W, 16), jnp.int32),
            pltpu.VMEM_SHARED((W, 512), jnp.int32),
        ],
    )(a_flat, gtc, wh, gpos, gneg, logtab)

    offset = off_flat.reshape(4, NT)[:, :N0].T
    return offset, label[:N0]


# speedup vs baseline: 1.1383x; 1.0034x over previous
import functools
import numpy as np
import jax
import jax.numpy as jnp
from jax import lax
from jax.experimental import pallas as pl
from jax.experimental.pallas import tpu as pltpu
from jax.experimental.pallas import tpu_sc as plsc

N0 = 20000
W = 16
NA = 1280
NT = W * NA
NTAB = 24576
NCH = NA // 16
NEG_INF = np.float32(-np.inf)
IMAX = np.int32(2147483647)
IMIN = np.int32(-2147483648)


def _iota():
    return lax.iota(jnp.int32, 16)


def _sumi(vec):
    return jnp.sum(vec.astype(jnp.float32)).astype(jnp.int32)


def _ext(vec, idx):
    return _sumi(jnp.where(_iota() == idx, vec, 0))


def _ln(x):
    u = lax.bitcast_convert_type(x, jnp.int32)
    e = ((u >> 23) & 0xFF) - 127
    m = lax.bitcast_convert_type((u & 0x007FFFFF) | 0x3F800000, jnp.float32)
    big = m > jnp.float32(1.4142135)
    m = jnp.where(big, m * jnp.float32(0.5), m)
    e = (e + jnp.where(big, 1, 0)).astype(jnp.float32)
    z = (m - 1.0) / (m + 1.0)
    z2 = z * z
    p = z * (2.0 + z2 * (2.0 / 3.0 + z2 * (0.4 + z2 * (2.0 / 7.0 + z2 * (2.0 / 9.0)))))
    return e * jnp.float32(0.6931471805599453) + p


def _orderable(k):
    u = lax.bitcast_convert_type(k, jnp.int32)
    return u ^ ((u >> 31) & 0x7FFFFFFF)


def _body(a_hbm, gtc_hbm, wh_hbm, gpos_hbm, gneg_hbm, logt_hbm,
          off_hbm, lab_hbm,
          av, maskv, lrankv, pamaxv, pargv, iouv, labelv, keyv, selv,
          gtcv, whv, gmaxv, gtmaxlocv, cntv, histv, gslv, logtv, offv,
          histall, gmaxall, cntall,
          sh_gtmax, sh_cnt, sh_hist):
    wid = lax.axis_index("s")
    base = wid * NA
    iot = _iota()
    onesi = jnp.ones((16,), jnp.int32)
    zerosi = jnp.zeros((16,), jnp.int32)

    for f in range(4):
        pltpu.sync_copy(a_hbm.at[pl.ds(f * NT + base, NA)], av.at[pl.ds(f * NA, NA)])
    pltpu.sync_copy(gtc_hbm, gtcv)
    pltpu.sync_copy(wh_hbm, whv)
    wv = whv[pl.ds(0, 16)]
    hv = whv[pl.ds(16, 16)]

    def p1_chunk(c, car):
        cmask, rbase = car
        ax1 = av[pl.ds(0 * NA + c * 16, 16)]
        ay1 = av[pl.ds(1 * NA + c * 16, 16)]
        ax2 = av[pl.ds(2 * NA + c * 16, 16)]
        ay2 = av[pl.ds(3 * NA + c * 16, 16)]
        mx = (ax1 >= 0.0) & (ay1 >= 0.0) & (ax2 <= wv) & (ay2 <= hv)
        mi = jnp.where(mx, 1, 0)
        maskv[pl.ds(c * 16, 16)] = mi
        mf = jnp.where(mx, 1.0, 0.0)
        cs = (plsc.cumsum(mf) - mf).astype(jnp.int32)
        lrankv[pl.ds(c * 16, 16)] = rbase + cs
        area_a = (ax2 - ax1) * (ay2 - ay1)

        def p1_gt(jq, icar):
            pamax, parg = icar
            for dj in range(8):
                j = jq * 8 + dj
                jf = jnp.full((16,), j, jnp.int32)
                gx1 = plsc.load_gather(gtcv, [zerosi, jf])
                gy1 = plsc.load_gather(gtcv, [onesi, jf])
                gx2 = plsc.load_gather(gtcv, [onesi + onesi, jf])
                gy2 = plsc.load_gather(gtcv, [onesi + onesi + onesi, jf])
                ix1 = jnp.maximum(ax1, gx1)
                iy1 = jnp.maximum(ay1, gy1)
                ix2 = jnp.minimum(ax2, gx2)
                iy2 = jnp.minimum(ay2, gy2)
                iw = jnp.maximum(ix2 - ix1, 0.0)
                ih = jnp.maximum(iy2 - iy1, 0.0)
                inter = iw * ih
                area_b = (gx2 - gx1) * (gy2 - gy1)
                iou = jnp.where(mx, inter / (area_a + area_b - inter), NEG_INF)
                iouv[pl.ds(j * NA + c * 16, 16)] = iou
                better = iou > pamax
                pamax = jnp.where(better, iou, pamax)
                parg = jnp.where(better, jf, parg)
            return pamax, parg

        pamax0 = jnp.full((16,), NEG_INF, jnp.float32)
        pamax, parg = lax.fori_loop(0, 4, p1_gt, (pamax0, zerosi))
        pamaxv[pl.ds(c * 16, 16)] = pamax
        pargv[pl.ds(c * 16, 16)] = parg
        nm = _sumi(mi)
        return cmask + nm, rbase + nm

    ninf16 = jnp.full((16,), NEG_INF, jnp.float32)
    cmask, _ = lax.fori_loop(
        0, NCH, p1_chunk, (jnp.int32(0), jnp.int32(0)))

    def rowmax_j(j, car):
        glo, ghi = car

        def rm_c(cq, acc):
            a0 = iouv[pl.ds(j * NA + cq * 64, 16)]
            a1 = iouv[pl.ds(j * NA + cq * 64 + 16, 16)]
            a2 = iouv[pl.ds(j * NA + cq * 64 + 32, 16)]
            a3 = iouv[pl.ds(j * NA + cq * 64 + 48, 16)]
            return jnp.maximum(acc, jnp.maximum(jnp.maximum(a0, a1),
                                                jnp.maximum(a2, a3)))

        s = jnp.max(lax.fori_loop(0, NCH // 4, rm_c, ninf16))
        sb = jnp.full((16,), s, jnp.float32)
        glo = jnp.where(iot == j, jnp.maximum(glo, sb), glo)
        ghi = jnp.where(iot == (j - 16), jnp.maximum(ghi, sb), ghi)
        return glo, ghi

    glo, ghi = lax.fori_loop(0, 32, rowmax_j, (ninf16, ninf16))
    gtmaxlocv[pl.ds(0, 16)] = glo
    gtmaxlocv[pl.ds(16, 16)] = ghi
    pltpu.sync_copy(gtmaxlocv, sh_gtmax.at[wid])
    plsc.subcore_barrier()

    pltpu.sync_copy(sh_gtmax, gmaxall)

    def gmax_w(w, car):
        glo, ghi = car
        return (jnp.maximum(glo, gmaxall[w, pl.ds(0, 16)]),
                jnp.maximum(ghi, gmaxall[w, pl.ds(16, 16)]))

    glo, ghi = lax.fori_loop(0, W, gmax_w, (ninf16, ninf16))
    gmaxv[pl.ds(0, 16)] = glo
    gmaxv[pl.ds(16, 16)] = ghi
    gmin = jnp.min(jnp.minimum(glo, ghi))

    def p2_chunk(c, car):
        cpos, cneg, spos, sneg = car
        pam = pamaxv[pl.ds(c * 16, 16)]
        mi = maskv[pl.ds(c * 16, 16)]
        mx = mi != 0
        lr = lrankv[pl.ds(c * 16, 16)]

        def p2_gt(jq, acc):
            e = []
            for dj in range(8):
                j = jq * 8 + dj
                gmb = plsc.load_gather(gmaxv, [jnp.full((16,), j, jnp.int32)])
                eq = (iouv[pl.ds(j * NA + c * 16, 16)] == gmb) & mx
                e.append(jnp.where(eq, 1, 0))
            return acc | ((e[0] | e[1]) | (e[2] | e[3])) | ((e[4] | e[5]) | (e[6] | e[7]))

        maybe = jnp.max(pam) >= gmin
        isgt = lax.cond(maybe,
                        lambda: lax.fori_loop(0, 4, p2_gt, zerosi),
                        lambda: zerosi) != 0
        one = isgt | (mx & (pam >= jnp.float32(0.7)))
        neg = mx & (pam <= jnp.float32(0.3)) & jnp.logical_not(one)
        lab = jnp.where(one, 1, jnp.where(neg, 0, -1))
        labelv[pl.ds(c * 16, 16)] = lab

        ax1 = av[pl.ds(0 * NA + c * 16, 16)]
        ay1 = av[pl.ds(1 * NA + c * 16, 16)]
        ax2 = av[pl.ds(2 * NA + c * 16, 16)]
        ay2 = av[pl.ds(3 * NA + c * 16, 16)]
        parg = pargv[pl.ds(c * 16, 16)]
        gx1 = plsc.load_gather(gtcv, [zerosi, parg])
        gy1 = plsc.load_gather(gtcv, [onesi, parg])
        gx2 = plsc.load_gather(gtcv, [onesi + onesi, parg])
        gy2 = plsc.load_gather(gtcv, [onesi + onesi + onesi, parg])
        aw = ax2 - ax1
        ah = ay2 - ay1
        acx = ax1 + 0.5 * aw
        acy = ay1 + 0.5 * ah
        gw = gx2 - gx1
        gh = gy2 - gy1
        gcx = gx1 + 0.5 * gw
        gcy = gy1 + 0.5 * gh
        zf = jnp.zeros((16,), jnp.float32)
        offv[pl.ds(0 * NA + c * 16, 16)] = jnp.where(mx, (gcx - acx) / aw, zf)
        offv[pl.ds(1 * NA + c * 16, 16)] = jnp.where(mx, (gcy - acy) / ah, zf)
        offv[pl.ds(2 * NA + c * 16, 16)] = jnp.where(mx, _ln(gw / aw), zf)
        offv[pl.ds(3 * NA + c * 16, 16)] = jnp.where(mx, _ln(gh / ah), zf)

        onei = jnp.where(one, 1, 0)
        negi = jnp.where(neg, 1, 0)
        return (cpos + _sumi(onei), cneg + _sumi(negi),
                spos + _sumi(jnp.where(one, lr, 0)),
                sneg + _sumi(jnp.where(neg, lr, 0)))

    cpos, cneg, spos, sneg = lax.fori_loop(
        0, NCH, p2_chunk, (jnp.int32(0),) * 4)
    for f in range(4):
        pltpu.sync_copy(offv.at[pl.ds(f * NA, NA)], off_hbm.at[pl.ds(f * NT + base, NA)])
    cv = jnp.where(iot == 0, cmask, 0)
    cv = jnp.where(iot == 1, cpos, cv)
    cv = jnp.where(iot == 2, cneg, cv)
    cv = jnp.where(iot == 3, spos, cv)
    cv = jnp.where(iot == 4, sneg, cv)
    cntv[pl.ds(0, 16)] = cv
    pltpu.sync_copy(cntv, sh_cnt.at[wid])
    plsc.subcore_barrier()

    pltpu.sync_copy(sh_cnt, cntall)

    def red_w(w, car):
        rb_all, rb_my, npos, posb, nneg, negb, rsp, rsn = car
        row = cntall[w, pl.ds(0, 16)]
        cm = _ext(row, 0)
        cp = _ext(row, 1)
        cn = _ext(row, 2)
        sp = _ext(row, 3)
        sn = _ext(row, 4)
        before = jnp.where(w < wid, 1, 0)
        return (rb_all + cm, rb_my + before * cm, npos + cp,
                posb + before * cp, nneg + cn, negb + before * cn,
                rsp + cp * rb_all + sp, rsn + cn * rb_all + sn)

    (_, rankbase, npos, posbase, nneg, negbase, rsumpos, rsumneg) = lax.fori_loop(
        0, W, red_w, (jnp.int32(0),) * 8)

    rb8 = pl.multiple_of((rankbase >> 3) << 3, 8)
    rrem = rankbase - rb8
    pltpu.sync_copy(logt_hbm.at[pl.ds(rb8, NA + 8)], logtv)

    Kp = jnp.int32(128)
    condp = (npos > Kp) & (rsumpos > 0)
    Kn = jnp.int32(256) - jnp.where(condp, Kp, npos)
    condn = (nneg > Kn) & (rsumneg > 0)
    cpi = jnp.where(condp, 1, 0)
    cni = jnp.where(condn, 1, 0)

    pb8 = pl.multiple_of((posbase >> 3) << 3, 8)
    prem = posbase - pb8
    nb8 = pl.multiple_of((negbase >> 3) << 3, 8)
    nrem = negbase - nb8
    pltpu.sync_copy(gpos_hbm.at[pl.ds(pb8, NA + 8)], gslv.at[pl.ds(0, NA + 8)])
    pltpu.sync_copy(gneg_hbm.at[pl.ds(nb8, NA + 8)], gslv.at[pl.ds(NA + 8, NA + 8)])

    def key_chunk(c, car):
        jbp, jbn = car
        lab = labelv[pl.ds(c * 16, 16)]
        lg = plsc.load_gather(logtv, [lrankv[pl.ds(c * 16, 16)] + rrem])
        selp = lab == 1
        spf = jnp.where(selp, 1.0, 0.0)
        jlp = jbp + (plsc.cumsum(spf) - spf).astype(jnp.int32)
        gvp = plsc.load_gather(gslv, [jlp + prem])
        vp = _orderable(gvp + lg)
        actp = selp & (cpi != 0)
        keyv[pl.ds(c * 16, 16)] = jnp.where(actp, vp, IMAX)
        selv[pl.ds(c * 16, 16)] = jnp.where(actp, 1, 0)
        seln = lab == 0
        snf = jnp.where(seln, 1.0, 0.0)
        jln = jbn + (plsc.cumsum(snf) - snf).astype(jnp.int32)
        gvn = plsc.load_gather(gslv, [jln + (NA + 8 + nrem)])
        vn = _orderable(gvn + lg)
        actn = seln & (cni != 0)
        keyv[pl.ds(NA + c * 16, 16)] = jnp.where(actn, vn, IMAX)
        selv[pl.ds(NA + c * 16, 16)] = jnp.where(actn, 1, 0)
        return jbp + _sumi(jnp.where(selp, 1, 0)), jbn + _sumi(jnp.where(seln, 1, 0))

    lax.fori_loop(0, NCH, key_chunk, (jnp.int32(0), jnp.int32(0)))

    def radix_round(r, pc):
        Pp, Cp, Pn, Cn = pc
        shift = 24 - 8 * r

        def zero_h(h, _):
            histv[pl.ds(h * 16, 16)] = zerosi
            return 0

        lax.fori_loop(0, 32, zero_h, 0)

        def hist_chunk(c, _):
            vp = keyv[pl.ds(c * 16, 16)]
            wp = vp ^ IMIN
            digp = lax.shift_right_logical(wp, shift) & 255
            hbp = lax.shift_right_logical(
                lax.shift_right_logical(wp, shift + 7), 1)
            actp = (selv[pl.ds(c * 16, 16)] != 0) & (hbp == Pp)
            plsc.addupdate_scatter(histv, [digp], onesi, mask=actp)
            vn = keyv[pl.ds(NA + c * 16, 16)]
            wn = vn ^ IMIN
            dign = (lax.shift_right_logical(wn, shift) & 255) + 256
            hbn = lax.shift_right_logical(
                lax.shift_right_logical(wn, shift + 7), 1)
            actn = (selv[pl.ds(NA + c * 16, 16)] != 0) & (hbn == Pn)
            plsc.addupdate_scatter(histv, [dign], onesi, mask=actn)
            return 0

        lax.fori_loop(0, NCH, hist_chunk, 0)
        pltpu.sync_copy(histv, sh_hist.at[wid])
        plsc.subcore_barrier()

        pltpu.sync_copy(sh_hist, histall)

        def merge_w(w, _):
            def merge_h(h, _):
                cur = jnp.where(w == 0, zerosi, histv[pl.ds(h * 16, 16)])
                histv[pl.ds(h * 16, 16)] = cur + histall[w, pl.ds(h * 16, 16)]
                return 0

            return lax.fori_loop(0, 32, merge_h, 0)

        lax.fori_loop(0, W, merge_w, 0)

        def find_digit(K, C, off):
            K1f = (K - C).astype(jnp.float32)

            def find_h(h, car):
                cum, found, bdig, cexcl = car
                accf = histv[pl.ds(off + h * 16, 16)].astype(jnp.float32)
                csf = plsc.cumsum(accf)
                cand = (cum.astype(jnp.float32) + csf) >= K1f
                idx = jnp.min(jnp.where(cand, iot.astype(jnp.float32), 16.0)).astype(jnp.int32)
                hit = (found == 0) & (idx < 16)
                csi = jnp.sum(jnp.where(iot == idx, csf - accf, 0.0)).astype(jnp.int32)
                bdig = jnp.where(hit, h * 16 + idx, bdig)
                cexcl = jnp.where(hit, cum + csi, cexcl)
                found = jnp.where(hit, 1, found)
                cum = cum + jnp.sum(jnp.where(iot == 15, csf, 0.0)).astype(jnp.int32)
                return cum, found, bdig, cexcl

            _, _, bdig, cexcl = lax.fori_loop(
                0, 16, find_h, (jnp.int32(0),) * 4)
            return bdig, cexcl

        bp, cep = find_digit(Kp, Cp, 0)
        bn, cen = find_digit(Kn, Cn, 256)
        plsc.subcore_barrier()
        return (Pp << 8) | bp, Cp + cep, (Pn << 8) | bn, Cn + cen

    Pp, _, Pn, _ = lax.fori_loop(
        0, 4, radix_round, (jnp.int32(0),) * 4)
    Tp = Pp ^ IMIN
    Tn = Pn ^ IMIN

    def apply_chunk(c, _):
        lab = labelv[pl.ds(c * 16, 16)]
        rmp = (selv[pl.ds(c * 16, 16)] != 0) & (keyv[pl.ds(c * 16, 16)] > Tp)
        rmn = (selv[pl.ds(NA + c * 16, 16)] != 0) & (keyv[pl.ds(NA + c * 16, 16)] > Tn)
        labelv[pl.ds(c * 16, 16)] = jnp.where(rmp | rmn, -1, lab)
        return 0

    lax.fori_loop(0, NCH, apply_chunk, 0)

    pltpu.sync_copy(labelv, lab_hbm.at[pl.ds(base, NA)])


_LOGTAB = np.full((NTAB,), 0.0, np.float32)
_LOGTAB[0] = -np.inf
_LOGTAB[1:] = np.log(np.arange(1, NTAB, dtype=np.float32))


@jax.jit
def kernel(gt_bbox, anchors, img_size):
    f32 = jnp.float32
    a_pad = jnp.concatenate(
        [anchors.astype(f32), jnp.full((NT - N0, 4), -1.0, f32)], axis=0)
    a_flat = a_pad.T.reshape(-1)
    gtc = gt_bbox.astype(f32).T
    w = img_size[1].astype(f32)
    h = img_size[0].astype(f32)
    wh = jnp.concatenate([jnp.full((16,), w, f32), jnp.full((16,), h, f32)])
    key = jax.random.key(42)
    kpos, kneg = jax.random.split(key)
    gpos = jax.random.gumbel(kpos, (N0,), f32)
    gneg = jax.random.gumbel(kneg, (N0,), f32)
    pad = jnp.zeros((NTAB - N0,), f32)
    gpos = jnp.concatenate([gpos, pad])
    gneg = jnp.concatenate([gneg, pad])
    logtab = jnp.asarray(_LOGTAB)

    mesh = plsc.VectorSubcoreMesh(
        core_axis_name="c", subcore_axis_name="s", num_cores=1)
    off_flat, label = pl.kernel(
        _body,
        out_type=[jax.ShapeDtypeStruct((4 * NT,), jnp.float32),
                  jax.ShapeDtypeStruct((NT,), jnp.int32)],
        mesh=mesh,
        compiler_params=pltpu.CompilerParams(needs_layout_passes=False),
        scratch_types=[
            pltpu.VMEM((4 * NA,), jnp.float32),
            pltpu.VMEM((NA,), jnp.int32),
            pltpu.VMEM((NA,), jnp.int32),
            pltpu.VMEM((NA,), jnp.float32),
            pltpu.VMEM((NA,), jnp.int32),
            pltpu.VMEM((32 * NA,), jnp.float32),
            pltpu.VMEM((NA,), jnp.int32),
            pltpu.VMEM((2 * NA,), jnp.int32),
            pltpu.VMEM((2 * NA,), jnp.int32),
            pltpu.VMEM((4, 32), jnp.float32),
            pltpu.VMEM((32,), jnp.float32),
            pltpu.VMEM((32,), jnp.float32),
            pltpu.VMEM((32,), jnp.float32),
            pltpu.VMEM((16,), jnp.int32),
            pltpu.VMEM((512,), jnp.int32),
            pltpu.VMEM((2 * (NA + 8),), jnp.float32),
            pltpu.VMEM((NA + 8,), jnp.float32),
            pltpu.VMEM((4 * NA,), jnp.float32),
            pltpu.VMEM((W, 512), jnp.int32),
            pltpu.VMEM((W, 32), jnp.float32),
            pltpu.VMEM((W, 16), jnp.int32),
            pltpu.VMEM_SHARED((W, 32), jnp.float32),
            pltpu.VMEM_SHARED((W, 16), jnp.int32),
            pltpu.VMEM_SHARED((W, 512), jnp.int32),
        ],
    )(a_flat, gtc, wh, gpos, gneg, logtab)

    offset = off_flat.reshape(4, NT)[:, :N0].T
    return offset, label[:N0]
